# baseline (speedup 1.0000x reference)
import functools
import math

import jax
import jax.numpy as jnp
import numpy as np
from jax.experimental import pallas as pl
from jax.experimental.pallas import tpu as pltpu

B, T, D = 1, 2048, 768
HQ, HKV = 12, 4
G = HQ // HKV
DH = 64
KS, STRIDE = 32, 16
BS = 64
TOPN = 8
WINDOW = 512
THETA = 10000.0

TC = (T - KS) // STRIDE + 1
TCP = 128
NBLK = T // BS
QB = 256
NQ = T // QB
KB = 256
NEG = -1e30
SCALE = 1.0 / math.sqrt(DH)

def _p_swap(n_heads):
    p1 = np.zeros((DH, DH), np.float32)
    p1[np.arange(32), np.arange(32) + 32] = 1.0
    p1[np.arange(32) + 32, np.arange(32)] = 1.0
    out = np.zeros((n_heads * DH, n_heads * DH), np.float32)
    for h in range(n_heads):
        out[h * DH:(h + 1) * DH, h * DH:(h + 1) * DH] = p1
    return jnp.asarray(out)


def _rope_tables(pos, n_heads):
    inv = 1.0 / (THETA ** (np.arange(0, DH, 2, dtype=np.float32) / DH))
    ang = pos.astype(np.float32)[:, None] * inv[None, :]
    c = np.cos(ang)
    s = np.sin(ang)
    c64 = np.concatenate([c, c], axis=1)
    s64 = np.concatenate([-s, s], axis=1)
    return (jnp.asarray(np.tile(c64, (1, n_heads))),
            jnp.asarray(np.tile(s64, (1, n_heads))))


_P64 = _p_swap(1)
_P192 = _p_swap(G)
_P256 = _p_swap(HKV)
_CQ192, _SQ192 = _rope_tables(np.arange(T), G)
_CK64, _SK64 = _rope_tables(np.arange(T), 1)
_pc = np.arange(TCP) * STRIDE
_CC256, _SC256 = _rope_tables(_pc, HKV)

_SH = np.zeros((TCP, TCP), np.float32)
_SH[np.arange(TCP - 1), np.arange(TCP - 1) + 1] = 1.0
_SH = jnp.asarray(_SH)

_M = np.zeros((TCP, NBLK), np.float32)
for _c in range(TC):
    _M[_c, (_c * STRIDE) // BS] = 1.0
_M = jnp.asarray(_M)

_E2048 = np.zeros((NBLK, T), np.float32)
for _b in range(NBLK):
    _E2048[_b, _b * BS:(_b + 1) * BS] = 1.0
_E2048 = jnp.asarray(_E2048)


def _dot(a, b, trans_b=False):
    dn = (((1,), (1 if trans_b else 0,)), ((), ()))
    return jax.lax.dot_general(a.astype(jnp.bfloat16), b.astype(jnp.bfloat16),
                               dn, preferred_element_type=jnp.float32)


def _dotx(a, b, trans_b=False):
    dn = (((1,), (1 if trans_b else 0,)), ((), ()))
    return jax.lax.dot_general(a, b, dn, preferred_element_type=jnp.float32,
                               precision=jax.lax.Precision.HIGHEST)


def _bf(x):
    return x.astype(jnp.bfloat16).astype(jnp.float32)


def _proj_kernel(x_ref, w_ref, o_ref):
    j = pl.program_id(1)
    r = _dot(x_ref[...], w_ref[...])
    o_ref[...] = jnp.where(j == 10, jax.nn.sigmoid(r), r)


def _cmp_kernel(k2_ref, v2_ref, wk_ref, wv_ref, pe_ref,
                sh_ref, cc_ref, sc_ref, p256_ref, kc_ref, vc_ref):
    ak = jnp.zeros((TCP, HKV * DH), jnp.float32)
    bk = jnp.zeros((TCP, HKV * DH), jnp.float32)
    av = jnp.zeros((TCP, HKV * DH), jnp.float32)
    bv = jnp.zeros((TCP, HKV * DH), jnp.float32)
    wkb = _bf(wk_ref[...])
    wvb = _bf(wv_ref[...])
    for j in range(STRIDE):
        ka = _bf(k2_ref[:, j, :] + pe_ref[j, :])
        kb = _bf(k2_ref[:, j, :] + pe_ref[j + STRIDE, :])
        va = _bf(v2_ref[:, j, :] + pe_ref[j, :])
        vb = _bf(v2_ref[:, j, :] + pe_ref[j + STRIDE, :])
        ak += ka * wkb[j, :]
        bk += kb * wkb[j + STRIDE, :]
        av += va * wvb[j, :]
        bv += vb * wvb[j + STRIDE, :]
    kc = ak + _dotx(sh_ref[...], bk)
    vc = av + _dotx(sh_ref[...], bv)
    kc_ref[...] = kc * cc_ref[...] + _dotx(kc, p256_ref[...]) * sc_ref[...]
    vc_ref[...] = vc


def _cmpattn_kernel(q_ref, kc_ref, vc_ref, cq_ref, sq_ref, p192_ref, m_ref,
                    o_ref, bm_ref):
    i = pl.program_id(1)
    q = q_ref[0]
    qr = q * cq_ref[...] + _dotx(q, p192_ref[...]) * sq_ref[...]
    trow = (i * QB + jax.lax.broadcasted_iota(jnp.int32, (QB, 1), 0))
    ccol = jax.lax.broadcasted_iota(jnp.int32, (QB, TCP), 1)
    maskf = ((ccol * STRIDE + (KS - 1) <= trow) & (ccol < TC)).astype(jnp.float32)
    kc = kc_ref[0]
    vc = vc_ref[0]
    psum = jnp.zeros((QB, TCP), jnp.float32)
    for g in range(G):
        qg = qr[:, g * DH:(g + 1) * DH]
        sc = _dot(qg, kc, trans_b=True) * SCALE
        scm = jnp.where(maskf > 0, sc, NEG)
        m = jnp.max(scm, axis=1, keepdims=True)
        p = jnp.exp(scm - m) * maskf
        denom = jnp.maximum(jnp.sum(p, axis=1, keepdims=True), 1e-9)
        p = p / denom
        o_ref[0, :, g * DH:(g + 1) * DH] = _dot(p, vc)
        psum += p
    bscore = _dot(psum, m_ref[...])
    qblk = trow // BS
    nio = jax.lax.broadcasted_iota(jnp.int32, (QB, NBLK), 1)
    forced = (nio == 0) | (nio == qblk) | (nio == qblk - 1)
    valid = nio <= qblk
    cur = jnp.where(valid, bscore + forced.astype(jnp.float32) * 1e4, NEG)
    niof = nio.astype(jnp.float32)
    bmask = jnp.zeros((QB, NBLK), jnp.float32)
    for _ in range(TOPN):
        mx = jnp.max(cur, axis=1, keepdims=True)
        idx = jnp.min(jnp.where(cur == mx, niof, 1e9), axis=1, keepdims=True)
        first = niof == idx
        bmask = bmask + first.astype(jnp.float32) * (mx > -1e20).astype(jnp.float32)
        cur = jnp.where(first, -1e38, cur)
    bm_ref[0, :, :] = bmask


def _main_kernel(q_ref, k_ref, v_ref, cq_ref, sq_ref, ck_ref, sk_ref,
                 p192_ref, p64_ref, e_ref, bm_ref, oc_ref, g_ref, o_ref,
                 tok_ref):
    i = pl.program_id(1)
    q = q_ref[0]
    qr = q * cq_ref[...] + _dotx(q, p192_ref[...]) * sq_ref[...]
    trow = (i * QB + jax.lax.broadcasted_iota(jnp.int32, (QB, 1), 0))
    ciota = jax.lax.broadcasted_iota(jnp.int32, (QB, KB), 1)
    tok_ref[...] = _dot(bm_ref[0], e_ref[...])

    for g in range(G):
        qg = qr[:, g * DH:(g + 1) * DH]

        def body(j, carry):
            m_s, l_s, a_s, m_w, l_w, a_w = carry
            kb = k_ref[0, pl.ds(j * KB, KB), :]
            kbr = (kb * ck_ref[pl.ds(j * KB, KB), :]
                   + _dotx(kb, p64_ref[...]) * sk_ref[pl.ds(j * KB, KB), :])
            vb = v_ref[0, pl.ds(j * KB, KB), :]
            s = _dot(qg, kbr, trans_b=True) * SCALE
            scol = j * KB + ciota
            causalf = (scol <= trow).astype(jnp.float32)
            mslc = tok_ref[:, pl.ds(j * KB, KB)] * causalf
            scm = jnp.where(mslc > 0, s, NEG)
            m_n = jnp.maximum(m_s, jnp.max(scm, axis=1, keepdims=True))
            alpha = jnp.exp(m_s - m_n)
            p = jnp.exp(scm - m_n) * mslc
            l_s = l_s * alpha + jnp.sum(p, axis=1, keepdims=True)
            a_s = a_s * alpha + _dot(p, vb)
            m_s = m_n

            def swa(_):
                mswa = causalf * (trow - scol <= WINDOW).astype(jnp.float32)
                scw = jnp.where(mswa > 0, s, NEG)
                mw_n = jnp.maximum(m_w, jnp.max(scw, axis=1, keepdims=True))
                aw = jnp.exp(m_w - mw_n)
                pw = jnp.exp(scw - mw_n) * mswa
                return (mw_n, l_w * aw + jnp.sum(pw, axis=1, keepdims=True),
                        a_w * aw + _dot(pw, vb))

            m_w, l_w, a_w = jax.lax.cond(j >= i - 2, swa,
                                         lambda _: (m_w, l_w, a_w), None)
            return m_s, l_s, a_s, m_w, l_w, a_w

        init = (jnp.full((QB, 1), NEG), jnp.zeros((QB, 1)),
                jnp.zeros((QB, DH)),
                jnp.full((QB, 1), NEG), jnp.zeros((QB, 1)),
                jnp.zeros((QB, DH)))
        m_s, l_s, a_s, m_w, l_w, a_w = jax.lax.fori_loop(0, i + 1, body, init)
        o_slc = a_s / jnp.maximum(l_s, 1e-9)
        o_swa = a_w / jnp.maximum(l_w, 1e-9)
        gc = g_ref[0, :, 3 * g:3 * g + 1]
        gs = g_ref[0, :, 3 * g + 1:3 * g + 2]
        gw = g_ref[0, :, 3 * g + 2:3 * g + 3]
        oc = oc_ref[0, :, g * DH:(g + 1) * DH]
        o_ref[0, :, g * DH:(g + 1) * DH] = gc * oc + gs * o_slc + gw * o_swa


@jax.jit
def kernel(x, Wq, Wk, Wv, Wg, wk_pool, wv_pool, pe):
    x2 = x.reshape(T, D)
    wall = jnp.zeros((D, 11 * 128), jnp.float32)
    wall = wall.at[:, :768].set(Wq).at[:, 768:1024].set(Wk)
    wall = wall.at[:, 1024:1280].set(Wv).at[:, 1280:1316].set(Wg)

    proj = pl.pallas_call(
        _proj_kernel,
        grid=(NQ, 11),
        in_specs=[pl.BlockSpec((QB, D), lambda i, j: (i, 0)),
                  pl.BlockSpec((D, 128), lambda i, j: (0, j))],
        out_specs=pl.BlockSpec((QB, 128), lambda i, j: (i, j)),
        out_shape=jax.ShapeDtypeStruct((T, 11 * 128), jnp.float32),
    )(x2, wall)

    q = proj[:, :768]
    k = proj[:, 768:1024]
    v = proj[:, 1024:1280]
    g36 = proj[:, 1280:1316]
    qh = q.reshape(T, HKV, G * DH).transpose(1, 0, 2)
    kh = k.reshape(T, HKV, DH).transpose(1, 0, 2)
    vh = v.reshape(T, HKV, DH).transpose(1, 0, 2)
    garr = jnp.zeros((HKV, T, 16), jnp.float32).at[:, :, :9].set(
        g36.reshape(T, HKV, 9).transpose(1, 0, 2))

    wkvec = jnp.repeat(wk_pool.T, DH, axis=1)
    wvvec = jnp.repeat(wv_pool.T, DH, axis=1)
    pef = pe.transpose(1, 0, 2).reshape(KS, HKV * DH)

    k2 = k.reshape(T // STRIDE, STRIDE, HKV * DH)
    v2 = v.reshape(T // STRIDE, STRIDE, HKV * DH)
    full = lambda shape: pl.BlockSpec(shape, lambda *a: tuple(0 for _ in shape))
    kc, vc = pl.pallas_call(
        _cmp_kernel,
        grid=(1,),
        in_specs=[full((TCP, STRIDE, HKV * DH)), full((TCP, STRIDE, HKV * DH)),
                  full((KS, HKV * DH)), full((KS, HKV * DH)),
                  full((KS, HKV * DH)),
                  full((TCP, TCP)), full((TCP, HKV * DH)), full((TCP, HKV * DH)),
                  full((HKV * DH, HKV * DH))],
        out_specs=[full((TCP, HKV * DH)), full((TCP, HKV * DH))],
        out_shape=[jax.ShapeDtypeStruct((TCP, HKV * DH), jnp.float32),
                   jax.ShapeDtypeStruct((TCP, HKV * DH), jnp.float32)],
    )(k2, v2, wkvec, wvvec, pef, _SH, _CC256, _SC256, _P256)
    kch = kc.reshape(TCP, HKV, DH).transpose(1, 0, 2)
    vch = vc.reshape(TCP, HKV, DH).transpose(1, 0, 2)

    ocmp, bm = pl.pallas_call(
        _cmpattn_kernel,
        grid=(HKV, NQ),
        in_specs=[pl.BlockSpec((1, QB, G * DH), lambda h, i: (h, i, 0)),
                  pl.BlockSpec((1, TCP, DH), lambda h, i: (h, 0, 0)),
                  pl.BlockSpec((1, TCP, DH), lambda h, i: (h, 0, 0)),
                  pl.BlockSpec((QB, G * DH), lambda h, i: (i, 0)),
                  pl.BlockSpec((QB, G * DH), lambda h, i: (i, 0)),
                  pl.BlockSpec((G * DH, G * DH), lambda h, i: (0, 0)),
                  pl.BlockSpec((TCP, NBLK), lambda h, i: (0, 0))],
        out_specs=[pl.BlockSpec((1, QB, G * DH), lambda h, i: (h, i, 0)),
                   pl.BlockSpec((1, QB, NBLK), lambda h, i: (h, i, 0))],
        out_shape=[jax.ShapeDtypeStruct((HKV, T, G * DH), jnp.float32),
                   jax.ShapeDtypeStruct((HKV, T, NBLK), jnp.float32)],
    )(qh, kch, vch, _CQ192, _SQ192, _P192, _M)

    out = pl.pallas_call(
        _main_kernel,
        grid=(HKV, NQ),
        in_specs=[pl.BlockSpec((1, QB, G * DH), lambda h, i: (h, i, 0)),
                  pl.BlockSpec((1, T, DH), lambda h, i: (h, 0, 0)),
                  pl.BlockSpec((1, T, DH), lambda h, i: (h, 0, 0)),
                  pl.BlockSpec((QB, G * DH), lambda h, i: (i, 0)),
                  pl.BlockSpec((QB, G * DH), lambda h, i: (i, 0)),
                  pl.BlockSpec((T, DH), lambda h, i: (0, 0)),
                  pl.BlockSpec((T, DH), lambda h, i: (0, 0)),
                  pl.BlockSpec((G * DH, G * DH), lambda h, i: (0, 0)),
                  pl.BlockSpec((DH, DH), lambda h, i: (0, 0)),
                  pl.BlockSpec((NBLK, T), lambda h, i: (0, 0)),
                  pl.BlockSpec((1, QB, NBLK), lambda h, i: (h, i, 0)),
                  pl.BlockSpec((1, QB, G * DH), lambda h, i: (h, i, 0)),
                  pl.BlockSpec((1, QB, 16), lambda h, i: (h, i, 0))],
        out_specs=pl.BlockSpec((1, QB, G * DH), lambda h, i: (h, i, 0)),
        out_shape=jax.ShapeDtypeStruct((HKV, T, G * DH), jnp.float32),
        scratch_shapes=[pltpu.VMEM((QB, T), jnp.float32)],
    )(qh, kh, vh, _CQ192, _SQ192, _CK64, _SK64, _P192, _P64, _E2048,
      bm, ocmp, garr)

    return out.transpose(1, 0, 2).reshape(B, T, HQ * DH)


# speedup vs baseline: 1.3392x; 1.3392x over previous
import functools
import math

import jax
import jax.numpy as jnp
import numpy as np
from jax.experimental import pallas as pl
from jax.experimental.pallas import tpu as pltpu

B, T, D = 1, 2048, 768
HQ, HKV = 12, 4
G = HQ // HKV
DH = 64
KS, STRIDE = 32, 16
BS = 64
TOPN = 8
WINDOW = 512
THETA = 10000.0

TC = (T - KS) // STRIDE + 1
TCP = 128
NBLK = T // BS
QB = 256
NQ = T // QB
KB = 256
NEG = -1e30
SCALE = 1.0 / math.sqrt(DH)

def _p_swap(n_heads):
    p1 = np.zeros((DH, DH), np.float32)
    p1[np.arange(32), np.arange(32) + 32] = 1.0
    p1[np.arange(32) + 32, np.arange(32)] = 1.0
    out = np.zeros((n_heads * DH, n_heads * DH), np.float32)
    for h in range(n_heads):
        out[h * DH:(h + 1) * DH, h * DH:(h + 1) * DH] = p1
    return jnp.asarray(out)


def _rope_tables(pos, n_heads):
    inv = 1.0 / (THETA ** (np.arange(0, DH, 2, dtype=np.float32) / DH))
    ang = pos.astype(np.float32)[:, None] * inv[None, :]
    c = np.cos(ang)
    s = np.sin(ang)
    c64 = np.concatenate([c, c], axis=1)
    s64 = np.concatenate([-s, s], axis=1)
    return (jnp.asarray(np.tile(c64, (1, n_heads))),
            jnp.asarray(np.tile(s64, (1, n_heads))))


_P64 = _p_swap(1)
_P192 = _p_swap(G)
_P256 = _p_swap(HKV)
_CQ192, _SQ192 = _rope_tables(np.arange(T), G)
_CK64, _SK64 = _rope_tables(np.arange(T), 1)
_pc = np.arange(TCP) * STRIDE
_CC256, _SC256 = _rope_tables(_pc, HKV)

_SH = np.zeros((TCP, TCP), np.float32)
_SH[np.arange(TCP - 1), np.arange(TCP - 1) + 1] = 1.0
_SH = jnp.asarray(_SH)

_M = np.zeros((TCP, NBLK), np.float32)
for _c in range(TC):
    _M[_c, (_c * STRIDE) // BS] = 1.0
_M = jnp.asarray(_M)

_E2048 = np.zeros((NBLK, T), np.float32)
for _b in range(NBLK):
    _E2048[_b, _b * BS:(_b + 1) * BS] = 1.0
_E2048 = jnp.asarray(_E2048)

_CMADD = np.full((T, TCP), -1e30, np.float32)
for _c in range(TC):
    _CMADD[_c * STRIDE + KS - 1:, _c] = 0.0
_CMADD = jnp.asarray(_CMADD)

_tr = np.arange(QB)[:, None]
_cc = np.arange(KB)[None, :]
_wm = np.zeros((3 * QB, KB), np.float32)
_cm = np.zeros((3 * QB, KB), np.float32)
for _d in range(3):
    ok = (_cc <= _d * KB + _tr) & (_d * KB + _tr - _cc <= WINDOW)
    _wm[_d * QB:(_d + 1) * QB] = np.where(ok, 0.0, -1e30)
    if _d == 0:
        _cm[_d * QB:(_d + 1) * QB] = np.where(_cc <= _tr, 0.0, -1e30)
_WM = jnp.asarray(_wm)
_CM = jnp.asarray(_cm)


def _dot(a, b, trans_b=False):
    dn = (((1,), (1 if trans_b else 0,)), ((), ()))
    return jax.lax.dot_general(a.astype(jnp.bfloat16), b.astype(jnp.bfloat16),
                               dn, preferred_element_type=jnp.float32)


def _dotx(a, b, trans_b=False):
    dn = (((1,), (1 if trans_b else 0,)), ((), ()))
    return jax.lax.dot_general(a, b, dn, preferred_element_type=jnp.float32,
                               precision=jax.lax.Precision.HIGHEST)


def _bf(x):
    return x.astype(jnp.bfloat16).astype(jnp.float32)


def _proj_kernel(x_ref, w_ref, o_ref):
    j = pl.program_id(1)
    r = _dot(x_ref[...], w_ref[...])
    o_ref[...] = jnp.where(j == 10, jax.nn.sigmoid(r), r)


def _cmp_kernel(k2_ref, v2_ref, wk_ref, wv_ref, pe_ref,
                sh_ref, cc_ref, sc_ref, p256_ref, kc_ref, vc_ref):
    ak = jnp.zeros((TCP, HKV * DH), jnp.float32)
    bk = jnp.zeros((TCP, HKV * DH), jnp.float32)
    av = jnp.zeros((TCP, HKV * DH), jnp.float32)
    bv = jnp.zeros((TCP, HKV * DH), jnp.float32)
    wkb = _bf(wk_ref[...])
    wvb = _bf(wv_ref[...])
    for j in range(STRIDE):
        ka = _bf(k2_ref[:, j, :] + pe_ref[j, :])
        kb = _bf(k2_ref[:, j, :] + pe_ref[j + STRIDE, :])
        va = _bf(v2_ref[:, j, :] + pe_ref[j, :])
        vb = _bf(v2_ref[:, j, :] + pe_ref[j + STRIDE, :])
        ak += ka * wkb[j, :]
        bk += kb * wkb[j + STRIDE, :]
        av += va * wvb[j, :]
        bv += vb * wvb[j + STRIDE, :]
    kc = ak + _dotx(sh_ref[...], bk)
    vc = av + _dotx(sh_ref[...], bv)
    kc_ref[...] = kc * cc_ref[...] + _dotx(kc, p256_ref[...]) * sc_ref[...]
    vc_ref[...] = vc


def _cmpattn_kernel(q_ref, kc_ref, vc_ref, cq_ref, sq_ref, p192_ref, m_ref,
                    cm_ref, o_ref, bm_ref):
    i = pl.program_id(1)
    q = q_ref[0]
    qr = q * cq_ref[...] + _dotx(q, p192_ref[...]) * sq_ref[...]
    trow = (i * QB + jax.lax.broadcasted_iota(jnp.int32, (QB, 1), 0))
    cmadd = cm_ref[...]
    kc = kc_ref[0]
    vc = vc_ref[0]
    psum = jnp.zeros((QB, TCP), jnp.float32)
    for g in range(G):
        qg = qr[:, g * DH:(g + 1) * DH]
        scm = _dot(qg, kc, trans_b=True) * SCALE + cmadd
        m = jnp.maximum(jnp.max(scm, axis=1, keepdims=True), -1e28)
        p = jnp.exp(scm - m)
        denom = jnp.maximum(jnp.sum(p, axis=1, keepdims=True), 1e-9)
        p = p / denom
        o_ref[0, :, g * DH:(g + 1) * DH] = _dot(p, vc)
        psum += p
    bscore = _dot(psum, m_ref[...])
    qblk = trow // BS
    nio = jax.lax.broadcasted_iota(jnp.int32, (QB, NBLK), 1)
    forced = (nio == 0) | (nio == qblk) | (nio == qblk - 1)
    valid = nio <= qblk
    cur = jnp.where(valid, bscore + forced.astype(jnp.float32) * 1e4, NEG)
    niof = nio.astype(jnp.float32)
    bmask = jnp.zeros((QB, NBLK), jnp.float32)
    for _ in range(TOPN):
        mx = jnp.max(cur, axis=1, keepdims=True)
        idx = jnp.min(jnp.where(cur == mx, niof, 1e9), axis=1, keepdims=True)
        first = niof == idx
        bmask = bmask + first.astype(jnp.float32) * (mx > -1e20).astype(jnp.float32)
        cur = jnp.where(first, -1e38, cur)
    bm_ref[0, :, :] = bmask


def _main_kernel(q_ref, k_ref, v_ref, cq_ref, sq_ref, ck_ref, sk_ref,
                 p192_ref, p64_ref, e_ref, wm_ref, cmn_ref,
                 bm_ref, oc_ref, g_ref, o_ref,
                 tok_ref, kr_ref, vb_ref):
    i = pl.program_id(1)

    @pl.when(i == 0)
    def _prep():
        kb = k_ref[0]
        krf = kb * ck_ref[...] + _dotx(kb, p64_ref[...]) * sk_ref[...]
        kr_ref[...] = krf.astype(jnp.bfloat16)
        vb_ref[...] = v_ref[0].astype(jnp.bfloat16)

    q = q_ref[0]
    qr = (q * cq_ref[...] + _dotx(q, p192_ref[...]) * sq_ref[...]
          ).astype(jnp.bfloat16)
    tok_ref[...] = (_dot(bm_ref[0], e_ref[...]) - 1.0) * 1e30

    j0 = jnp.maximum(i - 2, 0)
    for g in range(G):
        qg = qr[:, g * DH:(g + 1) * DH]

        def far(j, carry):
            m_s, l_s, a_s = carry
            s = _dot(qg, kr_ref[pl.ds(j * KB, KB), :], trans_b=True) * SCALE
            scm = s + tok_ref[:, pl.ds(j * KB, KB)]
            m_n = jnp.maximum(m_s, jnp.max(scm, axis=1, keepdims=True))
            alpha = jnp.exp(m_s - m_n)
            p = jnp.exp(scm - m_n)
            l_s = l_s * alpha + jnp.sum(p, axis=1, keepdims=True)
            a_s = a_s * alpha + _dot(p, vb_ref[pl.ds(j * KB, KB), :])
            return m_n, l_s, a_s

        finit = (jnp.full((QB, 1), NEG), jnp.zeros((QB, 1)),
                 jnp.zeros((QB, DH)))
        m_s, l_s, a_s = jax.lax.fori_loop(0, j0, far, finit)

        def near(j, carry):
            m_s, l_s, a_s, m_w, l_w, a_w = carry
            d = i - j
            s = _dot(qg, kr_ref[pl.ds(j * KB, KB), :], trans_b=True) * SCALE
            vb = vb_ref[pl.ds(j * KB, KB), :]
            scm = (s + tok_ref[:, pl.ds(j * KB, KB)]
                   + cmn_ref[pl.ds(d * QB, QB), :])
            m_n = jnp.maximum(m_s, jnp.max(scm, axis=1, keepdims=True))
            alpha = jnp.exp(m_s - m_n)
            p = jnp.exp(scm - m_n)
            l_s = l_s * alpha + jnp.sum(p, axis=1, keepdims=True)
            a_s = a_s * alpha + _dot(p, vb)
            scw = s + wm_ref[pl.ds(d * QB, QB), :]
            mw_n = jnp.maximum(m_w, jnp.max(scw, axis=1, keepdims=True))
            aw = jnp.exp(m_w - mw_n)
            pw = jnp.exp(scw - mw_n)
            l_w = l_w * aw + jnp.sum(pw, axis=1, keepdims=True)
            a_w = a_w * aw + _dot(pw, vb)
            return m_n, l_s, a_s, mw_n, l_w, a_w

        ninit = (m_s, l_s, a_s,
                 jnp.full((QB, 1), NEG), jnp.zeros((QB, 1)),
                 jnp.zeros((QB, DH)))
        m_s, l_s, a_s, m_w, l_w, a_w = jax.lax.fori_loop(j0, i + 1, near, ninit)

        o_slc = a_s / jnp.maximum(l_s, 1e-9)
        o_swa = a_w / jnp.maximum(l_w, 1e-9)
        gc = g_ref[0, :, 3 * g:3 * g + 1]
        gs = g_ref[0, :, 3 * g + 1:3 * g + 2]
        gw = g_ref[0, :, 3 * g + 2:3 * g + 3]
        oc = oc_ref[0, :, g * DH:(g + 1) * DH]
        o_ref[0, :, g * DH:(g + 1) * DH] = gc * oc + gs * o_slc + gw * o_swa


@jax.jit
def kernel(x, Wq, Wk, Wv, Wg, wk_pool, wv_pool, pe):
    x2 = x.reshape(T, D)
    wall = jnp.zeros((D, 11 * 128), jnp.float32)
    wall = wall.at[:, :768].set(Wq).at[:, 768:1024].set(Wk)
    wall = wall.at[:, 1024:1280].set(Wv).at[:, 1280:1316].set(Wg)

    proj = pl.pallas_call(
        _proj_kernel,
        grid=(NQ, 11),
        in_specs=[pl.BlockSpec((QB, D), lambda i, j: (i, 0)),
                  pl.BlockSpec((D, 128), lambda i, j: (0, j))],
        out_specs=pl.BlockSpec((QB, 128), lambda i, j: (i, j)),
        out_shape=jax.ShapeDtypeStruct((T, 11 * 128), jnp.float32),
    )(x2, wall)

    q = proj[:, :768]
    k = proj[:, 768:1024]
    v = proj[:, 1024:1280]
    g36 = proj[:, 1280:1316]
    qh = q.reshape(T, HKV, G * DH).transpose(1, 0, 2)
    kh = k.reshape(T, HKV, DH).transpose(1, 0, 2)
    vh = v.reshape(T, HKV, DH).transpose(1, 0, 2)
    garr = jnp.zeros((HKV, T, 16), jnp.float32).at[:, :, :9].set(
        g36.reshape(T, HKV, 9).transpose(1, 0, 2))

    wkvec = jnp.repeat(wk_pool.T, DH, axis=1)
    wvvec = jnp.repeat(wv_pool.T, DH, axis=1)
    pef = pe.transpose(1, 0, 2).reshape(KS, HKV * DH)

    k2 = k.reshape(T // STRIDE, STRIDE, HKV * DH)
    v2 = v.reshape(T // STRIDE, STRIDE, HKV * DH)
    full = lambda shape: pl.BlockSpec(shape, lambda *a: tuple(0 for _ in shape))
    kc, vc = pl.pallas_call(
        _cmp_kernel,
        grid=(1,),
        in_specs=[full((TCP, STRIDE, HKV * DH)), full((TCP, STRIDE, HKV * DH)),
                  full((KS, HKV * DH)), full((KS, HKV * DH)),
                  full((KS, HKV * DH)),
                  full((TCP, TCP)), full((TCP, HKV * DH)), full((TCP, HKV * DH)),
                  full((HKV * DH, HKV * DH))],
        out_specs=[full((TCP, HKV * DH)), full((TCP, HKV * DH))],
        out_shape=[jax.ShapeDtypeStruct((TCP, HKV * DH), jnp.float32),
                   jax.ShapeDtypeStruct((TCP, HKV * DH), jnp.float32)],
    )(k2, v2, wkvec, wvvec, pef, _SH, _CC256, _SC256, _P256)
    kch = kc.reshape(TCP, HKV, DH).transpose(1, 0, 2)
    vch = vc.reshape(TCP, HKV, DH).transpose(1, 0, 2)

    ocmp, bm = pl.pallas_call(
        _cmpattn_kernel,
        grid=(HKV, NQ),
        in_specs=[pl.BlockSpec((1, QB, G * DH), lambda h, i: (h, i, 0)),
                  pl.BlockSpec((1, TCP, DH), lambda h, i: (h, 0, 0)),
                  pl.BlockSpec((1, TCP, DH), lambda h, i: (h, 0, 0)),
                  pl.BlockSpec((QB, G * DH), lambda h, i: (i, 0)),
                  pl.BlockSpec((QB, G * DH), lambda h, i: (i, 0)),
                  pl.BlockSpec((G * DH, G * DH), lambda h, i: (0, 0)),
                  pl.BlockSpec((TCP, NBLK), lambda h, i: (0, 0)),
                  pl.BlockSpec((QB, TCP), lambda h, i: (i, 0))],
        out_specs=[pl.BlockSpec((1, QB, G * DH), lambda h, i: (h, i, 0)),
                   pl.BlockSpec((1, QB, NBLK), lambda h, i: (h, i, 0))],
        out_shape=[jax.ShapeDtypeStruct((HKV, T, G * DH), jnp.float32),
                   jax.ShapeDtypeStruct((HKV, T, NBLK), jnp.float32)],
    )(qh, kch, vch, _CQ192, _SQ192, _P192, _M, _CMADD)

    out = pl.pallas_call(
        _main_kernel,
        grid=(HKV, NQ),
        in_specs=[pl.BlockSpec((1, QB, G * DH), lambda h, i: (h, i, 0)),
                  pl.BlockSpec((1, T, DH), lambda h, i: (h, 0, 0)),
                  pl.BlockSpec((1, T, DH), lambda h, i: (h, 0, 0)),
                  pl.BlockSpec((QB, G * DH), lambda h, i: (i, 0)),
                  pl.BlockSpec((QB, G * DH), lambda h, i: (i, 0)),
                  pl.BlockSpec((T, DH), lambda h, i: (0, 0)),
                  pl.BlockSpec((T, DH), lambda h, i: (0, 0)),
                  pl.BlockSpec((G * DH, G * DH), lambda h, i: (0, 0)),
                  pl.BlockSpec((DH, DH), lambda h, i: (0, 0)),
                  pl.BlockSpec((NBLK, T), lambda h, i: (0, 0)),
                  pl.BlockSpec((3 * QB, KB), lambda h, i: (0, 0)),
                  pl.BlockSpec((3 * QB, KB), lambda h, i: (0, 0)),
                  pl.BlockSpec((1, QB, NBLK), lambda h, i: (h, i, 0)),
                  pl.BlockSpec((1, QB, G * DH), lambda h, i: (h, i, 0)),
                  pl.BlockSpec((1, QB, 16), lambda h, i: (h, i, 0))],
        out_specs=pl.BlockSpec((1, QB, G * DH), lambda h, i: (h, i, 0)),
        out_shape=jax.ShapeDtypeStruct((HKV, T, G * DH), jnp.float32),
        scratch_shapes=[pltpu.VMEM((QB, T), jnp.float32),
                        pltpu.VMEM((T, DH), jnp.bfloat16),
                        pltpu.VMEM((T, DH), jnp.bfloat16)],
    )(qh, kh, vh, _CQ192, _SQ192, _CK64, _SK64, _P192, _P64, _E2048,
      _WM, _CM, bm, ocmp, garr)

    return out.transpose(1, 0, 2).reshape(B, T, HQ * DH)


# speedup vs baseline: 1.3660x; 1.0201x over previous
import functools
import math

import jax
import jax.numpy as jnp
import numpy as np
from jax.experimental import pallas as pl
from jax.experimental.pallas import tpu as pltpu

B, T, D = 1, 2048, 768
HQ, HKV = 12, 4
G = HQ // HKV
DH = 64
KS, STRIDE = 32, 16
BS = 64
TOPN = 8
WINDOW = 512
THETA = 10000.0

TC = (T - KS) // STRIDE + 1
TCP = 128
NBLK = T // BS
QB = 256
NQ = T // QB
KB = 256
NEG = -1e30
SCALE = 1.0 / math.sqrt(DH)

def _p_swap(n_heads):
    p1 = np.zeros((DH, DH), np.float32)
    p1[np.arange(32), np.arange(32) + 32] = 1.0
    p1[np.arange(32) + 32, np.arange(32)] = 1.0
    out = np.zeros((n_heads * DH, n_heads * DH), np.float32)
    for h in range(n_heads):
        out[h * DH:(h + 1) * DH, h * DH:(h + 1) * DH] = p1
    return jnp.asarray(out)


def _rope_tables(pos, n_heads):
    inv = 1.0 / (THETA ** (np.arange(0, DH, 2, dtype=np.float32) / DH))
    ang = pos.astype(np.float32)[:, None] * inv[None, :]
    c = np.cos(ang)
    s = np.sin(ang)
    c64 = np.concatenate([c, c], axis=1)
    s64 = np.concatenate([-s, s], axis=1)
    return (jnp.asarray(np.tile(c64, (1, n_heads))),
            jnp.asarray(np.tile(s64, (1, n_heads))))


_P64 = _p_swap(1)
_P192 = _p_swap(G)
_P256 = _p_swap(HKV)
_CQ192, _SQ192 = _rope_tables(np.arange(T), G)
_CK64, _SK64 = _rope_tables(np.arange(T), 1)
_pc = np.arange(TCP) * STRIDE
_CC256, _SC256 = _rope_tables(_pc, HKV)

_SH = np.zeros((TCP, TCP), np.float32)
_SH[np.arange(TCP - 1), np.arange(TCP - 1) + 1] = 1.0
_SH = jnp.asarray(_SH)

_M = np.zeros((TCP, NBLK), np.float32)
for _c in range(TC):
    _M[_c, (_c * STRIDE) // BS] = 1.0
_M = jnp.asarray(_M)

_E2048 = np.zeros((NBLK, T), np.float32)
for _b in range(NBLK):
    _E2048[_b, _b * BS:(_b + 1) * BS] = 1.0
_E2048 = jnp.asarray(_E2048)

_CMADD = np.full((T, TCP), -1e30, np.float32)
for _c in range(TC):
    _CMADD[_c * STRIDE + KS - 1:, _c] = 0.0
_CMADD = jnp.asarray(_CMADD)

_tr = np.arange(QB)[:, None]
_cc = np.arange(KB)[None, :]
_wm = np.zeros((3 * QB, KB), np.float32)
_cm = np.zeros((3 * QB, KB), np.float32)
for _d in range(3):
    ok = (_cc <= _d * KB + _tr) & (_d * KB + _tr - _cc <= WINDOW)
    _wm[_d * QB:(_d + 1) * QB] = np.where(ok, 0.0, -1e30)
    if _d == 0:
        _cm[_d * QB:(_d + 1) * QB] = np.where(_cc <= _tr, 0.0, -1e30)
_WM = jnp.asarray(_wm)
_CM = jnp.asarray(_cm)


def _dot(a, b, trans_b=False):
    dn = (((1,), (1 if trans_b else 0,)), ((), ()))
    return jax.lax.dot_general(a.astype(jnp.bfloat16), b.astype(jnp.bfloat16),
                               dn, preferred_element_type=jnp.float32)


def _dotx(a, b, trans_b=False):
    dn = (((1,), (1 if trans_b else 0,)), ((), ()))
    return jax.lax.dot_general(a, b, dn, preferred_element_type=jnp.float32,
                               precision=jax.lax.Precision.HIGHEST)


def _bf(x):
    return x.astype(jnp.bfloat16).astype(jnp.float32)


def _proj_kernel(x_ref, w_ref, o_ref):
    j = pl.program_id(1)
    r = _dot(x_ref[...], w_ref[...])
    o_ref[...] = jnp.where(j == 10, jax.nn.sigmoid(r), r)


def _cmp_kernel(k2_ref, v2_ref, wk_ref, wv_ref, pe_ref,
                sh_ref, cc_ref, sc_ref, p256_ref, kc_ref, vc_ref):
    ak = jnp.zeros((TCP, HKV * DH), jnp.float32)
    bk = jnp.zeros((TCP, HKV * DH), jnp.float32)
    av = jnp.zeros((TCP, HKV * DH), jnp.float32)
    bv = jnp.zeros((TCP, HKV * DH), jnp.float32)
    wkb = _bf(wk_ref[...])
    wvb = _bf(wv_ref[...])
    for j in range(STRIDE):
        ka = _bf(k2_ref[:, j, :] + pe_ref[j, :])
        kb = _bf(k2_ref[:, j, :] + pe_ref[j + STRIDE, :])
        va = _bf(v2_ref[:, j, :] + pe_ref[j, :])
        vb = _bf(v2_ref[:, j, :] + pe_ref[j + STRIDE, :])
        ak += ka * wkb[j, :]
        bk += kb * wkb[j + STRIDE, :]
        av += va * wvb[j, :]
        bv += vb * wvb[j + STRIDE, :]
    kc = ak + _dotx(sh_ref[...], bk)
    vc = av + _dotx(sh_ref[...], bv)
    kc_ref[...] = kc * cc_ref[...] + _dotx(kc, p256_ref[...]) * sc_ref[...]
    vc_ref[...] = vc


def _main_kernel(q_ref, k_ref, v_ref, kc_ref, vc_ref, cq_ref, sq_ref,
                 ck_ref, sk_ref, p192_ref, p64_ref, m_ref, cma_ref,
                 e_ref, wm_ref, cmn_ref, g_ref, o_ref,
                 tok_ref, kr_ref, vb_ref):
    i = pl.program_id(1)

    @pl.when(i == 0)
    def _prep():
        kb = k_ref[0]
        krf = kb * ck_ref[...] + _dotx(kb, p64_ref[...]) * sk_ref[...]
        kr_ref[...] = krf.astype(jnp.bfloat16)
        vb_ref[...] = v_ref[0].astype(jnp.bfloat16)

    q = q_ref[0]
    qr = (q * cq_ref[...] + _dotx(q, p192_ref[...]) * sq_ref[...]
          ).astype(jnp.bfloat16)

    cmadd = cma_ref[...]
    kc = kc_ref[0]
    vc = vc_ref[0]
    psum = jnp.zeros((QB, TCP), jnp.float32)
    ocmp = []
    for g in range(G):
        qg = qr[:, g * DH:(g + 1) * DH]
        scm = _dot(qg, kc, trans_b=True) * SCALE + cmadd
        m = jnp.maximum(jnp.max(scm, axis=1, keepdims=True), -1e28)
        p = jnp.exp(scm - m)
        denom = jnp.maximum(jnp.sum(p, axis=1, keepdims=True), 1e-9)
        p = p / denom
        ocmp.append(_dot(p, vc))
        psum += p

    bscore = _dot(psum, m_ref[...])
    trow = (i * QB + jax.lax.broadcasted_iota(jnp.int32, (QB, 1), 0))
    qblk = trow // BS
    nio = jax.lax.broadcasted_iota(jnp.int32, (QB, NBLK), 1)
    forced = (nio == 0) | (nio == qblk) | (nio == qblk - 1)
    valid = nio <= qblk
    cur = jnp.where(valid, bscore + forced.astype(jnp.float32) * 1e4, NEG)
    niof = nio.astype(jnp.float32)
    bmask = jnp.zeros((QB, NBLK), jnp.float32)
    for _ in range(TOPN):
        mx = jnp.max(cur, axis=1, keepdims=True)
        idx = jnp.min(jnp.where(cur == mx, niof, 1e9), axis=1, keepdims=True)
        first = niof == idx
        bmask = bmask + first.astype(jnp.float32) * (mx > -1e20).astype(jnp.float32)
        cur = jnp.where(first, -1e38, cur)

    tok_ref[...] = (_dot(bmask, e_ref[...]) - 1.0) * 1e30

    j0 = jnp.maximum(i - 2, 0)
    for g in range(G):
        qg = qr[:, g * DH:(g + 1) * DH]

        def far(j, carry):
            m_s, l_s, a_s = carry
            s = _dot(qg, kr_ref[pl.ds(j * KB, KB), :], trans_b=True) * SCALE
            scm = s + tok_ref[:, pl.ds(j * KB, KB)]
            m_n = jnp.maximum(m_s, jnp.max(scm, axis=1, keepdims=True))
            alpha = jnp.exp(m_s - m_n)
            p = jnp.exp(scm - m_n)
            l_s = l_s * alpha + jnp.sum(p, axis=1, keepdims=True)
            a_s = a_s * alpha + _dot(p, vb_ref[pl.ds(j * KB, KB), :])
            return m_n, l_s, a_s

        finit = (jnp.full((QB, 1), NEG), jnp.zeros((QB, 1)),
                 jnp.zeros((QB, DH)))
        m_s, l_s, a_s = jax.lax.fori_loop(0, j0, far, finit)

        def near(j, carry):
            m_s, l_s, a_s, m_w, l_w, a_w = carry
            d = i - j
            s = _dot(qg, kr_ref[pl.ds(j * KB, KB), :], trans_b=True) * SCALE
            vb = vb_ref[pl.ds(j * KB, KB), :]
            scm = (s + tok_ref[:, pl.ds(j * KB, KB)]
                   + cmn_ref[pl.ds(d * QB, QB), :])
            m_n = jnp.maximum(m_s, jnp.max(scm, axis=1, keepdims=True))
            alpha = jnp.exp(m_s - m_n)
            p = jnp.exp(scm - m_n)
            l_s = l_s * alpha + jnp.sum(p, axis=1, keepdims=True)
            a_s = a_s * alpha + _dot(p, vb)
            scw = s + wm_ref[pl.ds(d * QB, QB), :]
            mw_n = jnp.maximum(m_w, jnp.max(scw, axis=1, keepdims=True))
            aw = jnp.exp(m_w - mw_n)
            pw = jnp.exp(scw - mw_n)
            l_w = l_w * aw + jnp.sum(pw, axis=1, keepdims=True)
            a_w = a_w * aw + _dot(pw, vb)
            return m_n, l_s, a_s, mw_n, l_w, a_w

        ninit = (m_s, l_s, a_s,
                 jnp.full((QB, 1), NEG), jnp.zeros((QB, 1)),
                 jnp.zeros((QB, DH)))
        m_s, l_s, a_s, m_w, l_w, a_w = jax.lax.fori_loop(j0, i + 1, near, ninit)

        o_slc = a_s / jnp.maximum(l_s, 1e-9)
        o_swa = a_w / jnp.maximum(l_w, 1e-9)
        gc = g_ref[0, :, 3 * g:3 * g + 1]
        gs = g_ref[0, :, 3 * g + 1:3 * g + 2]
        gw = g_ref[0, :, 3 * g + 2:3 * g + 3]
        o_ref[0, :, g * DH:(g + 1) * DH] = (gc * ocmp[g] + gs * o_slc
                                            + gw * o_swa)


@jax.jit
def kernel(x, Wq, Wk, Wv, Wg, wk_pool, wv_pool, pe):
    x2 = x.reshape(T, D)
    wall = jnp.zeros((D, 11 * 128), jnp.float32)
    wall = wall.at[:, :768].set(Wq).at[:, 768:1024].set(Wk)
    wall = wall.at[:, 1024:1280].set(Wv).at[:, 1280:1316].set(Wg)

    proj = pl.pallas_call(
        _proj_kernel,
        grid=(NQ, 11),
        in_specs=[pl.BlockSpec((QB, D), lambda i, j: (i, 0)),
                  pl.BlockSpec((D, 128), lambda i, j: (0, j))],
        out_specs=pl.BlockSpec((QB, 128), lambda i, j: (i, j)),
        out_shape=jax.ShapeDtypeStruct((T, 11 * 128), jnp.float32),
    )(x2, wall)

    q = proj[:, :768]
    k = proj[:, 768:1024]
    v = proj[:, 1024:1280]
    g36 = proj[:, 1280:1316]
    qh = q.reshape(T, HKV, G * DH).transpose(1, 0, 2)
    kh = k.reshape(T, HKV, DH).transpose(1, 0, 2)
    vh = v.reshape(T, HKV, DH).transpose(1, 0, 2)
    garr = jnp.zeros((HKV, T, 16), jnp.float32).at[:, :, :9].set(
        g36.reshape(T, HKV, 9).transpose(1, 0, 2))

    wkvec = jnp.repeat(wk_pool.T, DH, axis=1)
    wvvec = jnp.repeat(wv_pool.T, DH, axis=1)
    pef = pe.transpose(1, 0, 2).reshape(KS, HKV * DH)

    k2 = k.reshape(T // STRIDE, STRIDE, HKV * DH)
    v2 = v.reshape(T // STRIDE, STRIDE, HKV * DH)
    full = lambda shape: pl.BlockSpec(shape, lambda *a: tuple(0 for _ in shape))
    kc, vc = pl.pallas_call(
        _cmp_kernel,
        grid=(1,),
        in_specs=[full((TCP, STRIDE, HKV * DH)), full((TCP, STRIDE, HKV * DH)),
                  full((KS, HKV * DH)), full((KS, HKV * DH)),
                  full((KS, HKV * DH)),
                  full((TCP, TCP)), full((TCP, HKV * DH)), full((TCP, HKV * DH)),
                  full((HKV * DH, HKV * DH))],
        out_specs=[full((TCP, HKV * DH)), full((TCP, HKV * DH))],
        out_shape=[jax.ShapeDtypeStruct((TCP, HKV * DH), jnp.float32),
                   jax.ShapeDtypeStruct((TCP, HKV * DH), jnp.float32)],
    )(k2, v2, wkvec, wvvec, pef, _SH, _CC256, _SC256, _P256)
    kch = kc.reshape(TCP, HKV, DH).transpose(1, 0, 2)
    vch = vc.reshape(TCP, HKV, DH).transpose(1, 0, 2)

    out = pl.pallas_call(
        _main_kernel,
        grid=(HKV, NQ),
        in_specs=[pl.BlockSpec((1, QB, G * DH), lambda h, i: (h, i, 0)),
                  pl.BlockSpec((1, T, DH), lambda h, i: (h, 0, 0)),
                  pl.BlockSpec((1, T, DH), lambda h, i: (h, 0, 0)),
                  pl.BlockSpec((1, TCP, DH), lambda h, i: (h, 0, 0)),
                  pl.BlockSpec((1, TCP, DH), lambda h, i: (h, 0, 0)),
                  pl.BlockSpec((QB, G * DH), lambda h, i: (i, 0)),
                  pl.BlockSpec((QB, G * DH), lambda h, i: (i, 0)),
                  pl.BlockSpec((T, DH), lambda h, i: (0, 0)),
                  pl.BlockSpec((T, DH), lambda h, i: (0, 0)),
                  pl.BlockSpec((G * DH, G * DH), lambda h, i: (0, 0)),
                  pl.BlockSpec((DH, DH), lambda h, i: (0, 0)),
                  pl.BlockSpec((TCP, NBLK), lambda h, i: (0, 0)),
                  pl.BlockSpec((QB, TCP), lambda h, i: (i, 0)),
                  pl.BlockSpec((NBLK, T), lambda h, i: (0, 0)),
                  pl.BlockSpec((3 * QB, KB), lambda h, i: (0, 0)),
                  pl.BlockSpec((3 * QB, KB), lambda h, i: (0, 0)),
                  pl.BlockSpec((1, QB, 16), lambda h, i: (h, i, 0))],
        out_specs=pl.BlockSpec((1, QB, G * DH), lambda h, i: (h, i, 0)),
        out_shape=jax.ShapeDtypeStruct((HKV, T, G * DH), jnp.float32),
        scratch_shapes=[pltpu.VMEM((QB, T), jnp.float32),
                        pltpu.VMEM((T, DH), jnp.bfloat16),
                        pltpu.VMEM((T, DH), jnp.bfloat16)],
    )(qh, kh, vh, kch, vch, _CQ192, _SQ192, _CK64, _SK64, _P192, _P64,
      _M, _CMADD, _E2048, _WM, _CM, garr)

    return out.transpose(1, 0, 2).reshape(B, T, HQ * DH)


# speedup vs baseline: 1.4223x; 1.0412x over previous
import functools
import math

import jax
import jax.numpy as jnp
import numpy as np
from jax.experimental import pallas as pl
from jax.experimental.pallas import tpu as pltpu

B, T, D = 1, 2048, 768
HQ, HKV = 12, 4
G = HQ // HKV
DH = 64
KS, STRIDE = 32, 16
BS = 64
TOPN = 8
WINDOW = 512
THETA = 10000.0

TC = (T - KS) // STRIDE + 1
TCP = 128
NBLK = T // BS
QB = 256
NQ = T // QB
KB = 256
NEG = -1e30
SCALE = 1.0 / math.sqrt(DH)

def _p_swap(n_heads):
    p1 = np.zeros((DH, DH), np.float32)
    p1[np.arange(32), np.arange(32) + 32] = 1.0
    p1[np.arange(32) + 32, np.arange(32)] = 1.0
    out = np.zeros((n_heads * DH, n_heads * DH), np.float32)
    for h in range(n_heads):
        out[h * DH:(h + 1) * DH, h * DH:(h + 1) * DH] = p1
    return out


def _rope_tables(pos, n_heads):
    inv = 1.0 / (THETA ** (np.arange(0, DH, 2, dtype=np.float32) / DH))
    ang = pos.astype(np.float32)[:, None] * inv[None, :]
    c = np.cos(ang)
    s = np.sin(ang)
    c64 = np.concatenate([c, c], axis=1)
    s64 = np.concatenate([-s, s], axis=1)
    return np.tile(c64, (1, n_heads)), np.tile(s64, (1, n_heads))


_P64 = _p_swap(1)
_P192 = _p_swap(G)
_P256 = _p_swap(HKV)
_CQ192, _SQ192 = _rope_tables(np.arange(T), G)
_CK64, _SK64 = _rope_tables(np.arange(T), 1)
_pc = np.arange(TCP) * STRIDE
_CC256, _SC256 = _rope_tables(_pc, HKV)

_SH = np.zeros((TCP, TCP), np.float32)
_SH[np.arange(TCP - 1), np.arange(TCP - 1) + 1] = 1.0


_M = np.zeros((TCP, NBLK), np.float32)
for _c in range(TC):
    _M[_c, (_c * STRIDE) // BS] = 1.0


_E2048 = np.zeros((NBLK, T), np.float32)
for _b in range(NBLK):
    _E2048[_b, _b * BS:(_b + 1) * BS] = 1.0


_CMADD = np.full((T, TCP), -1e30, np.float32)
for _c in range(TC):
    _CMADD[_c * STRIDE + KS - 1:, _c] = 0.0


_tr = np.arange(QB)[:, None]
_cc = np.arange(KB)[None, :]
_wm = np.zeros((3 * QB, KB), np.float32)
_cm = np.zeros((3 * QB, KB), np.float32)
for _d in range(3):
    ok = (_cc <= _d * KB + _tr) & (_d * KB + _tr - _cc <= WINDOW)
    _wm[_d * QB:(_d + 1) * QB] = np.where(ok, 0.0, -1e30)
    if _d == 0:
        _cm[_d * QB:(_d + 1) * QB] = np.where(_cc <= _tr, 0.0, -1e30)
_WM = _wm
_CM = _cm


def _dot(a, b, trans_b=False):
    dn = (((1,), (1 if trans_b else 0,)), ((), ()))
    return jax.lax.dot_general(a.astype(jnp.bfloat16), b.astype(jnp.bfloat16),
                               dn, preferred_element_type=jnp.float32)


def _dotx(a, b, trans_b=False):
    dn = (((1,), (1 if trans_b else 0,)), ((), ()))
    return jax.lax.dot_general(a, b, dn, preferred_element_type=jnp.float32,
                               precision=jax.lax.Precision.HIGHEST)


def _bf(x):
    return x.astype(jnp.bfloat16).astype(jnp.float32)


def _proj_kernel(x_ref, w_ref, o_ref):
    j = pl.program_id(1)
    r = _dot(x_ref[...], w_ref[...])
    o_ref[...] = jnp.where(j == 10, jax.nn.sigmoid(r), r)


def _cmp_kernel(k2_ref, v2_ref, wk_ref, wv_ref, pe_ref,
                sh_ref, cc_ref, sc_ref, p256_ref, kc_ref, vc_ref):
    ak = jnp.zeros((TCP, HKV * DH), jnp.float32)
    bk = jnp.zeros((TCP, HKV * DH), jnp.float32)
    av = jnp.zeros((TCP, HKV * DH), jnp.float32)
    bv = jnp.zeros((TCP, HKV * DH), jnp.float32)
    wkb = _bf(wk_ref[...])
    wvb = _bf(wv_ref[...])
    for j in range(STRIDE):
        k2j = k2_ref[j]
        v2j = v2_ref[j]
        ka = _bf(k2j + pe_ref[j, :])
        kb = _bf(k2j + pe_ref[j + STRIDE, :])
        va = _bf(v2j + pe_ref[j, :])
        vb = _bf(v2j + pe_ref[j + STRIDE, :])
        ak += ka * wkb[j, :]
        bk += kb * wkb[j + STRIDE, :]
        av += va * wvb[j, :]
        bv += vb * wvb[j + STRIDE, :]
    kc = ak + _dotx(sh_ref[...], bk)
    vc = av + _dotx(sh_ref[...], bv)
    kc_ref[...] = kc * cc_ref[...] + _dotx(kc, p256_ref[...]) * sc_ref[...]
    vc_ref[...] = vc


def _main_kernel(q_ref, k_ref, v_ref, kc_ref, vc_ref, cq_ref, sq_ref,
                 ck_ref, sk_ref, p192_ref, p64_ref, m_ref, cma_ref,
                 e_ref, wm_ref, cmn_ref, g_ref, o_ref,
                 tok_ref, kr_ref, vb_ref):
    i = pl.program_id(1)

    @pl.when(i == 0)
    def _prep():
        kb = k_ref[0]
        krf = kb * ck_ref[...] + _dotx(kb, p64_ref[...]) * sk_ref[...]
        kr_ref[...] = krf.astype(jnp.bfloat16)
        vb_ref[...] = v_ref[0].astype(jnp.bfloat16)

    q = q_ref[0]
    qr = (q * cq_ref[...] + _dotx(q, p192_ref[...]) * sq_ref[...]
          ).astype(jnp.bfloat16)

    cmadd = cma_ref[...]
    kc = kc_ref[0]
    vc = vc_ref[0]
    psum = jnp.zeros((QB, TCP), jnp.float32)
    ocmp = []
    for g in range(G):
        qg = qr[:, g * DH:(g + 1) * DH]
        scm = _dot(qg, kc, trans_b=True) * SCALE + cmadd
        m = jnp.maximum(jnp.max(scm, axis=1, keepdims=True), -1e28)
        p = jnp.exp(scm - m)
        denom = jnp.maximum(jnp.sum(p, axis=1, keepdims=True), 1e-9)
        p = p / denom
        ocmp.append(_dot(p, vc))
        psum += p

    bscore = _dot(psum, m_ref[...])
    trow = (i * QB + jax.lax.broadcasted_iota(jnp.int32, (QB, 1), 0))
    qblk = trow // BS
    nio = jax.lax.broadcasted_iota(jnp.int32, (QB, NBLK), 1)
    forced = (nio == 0) | (nio == qblk) | (nio == qblk - 1)
    valid = nio <= qblk
    cur = jnp.where(valid, bscore + forced.astype(jnp.float32) * 1e4, NEG)
    niof = nio.astype(jnp.float32)
    bmask = jnp.zeros((QB, NBLK), jnp.float32)
    for _ in range(TOPN):
        mx = jnp.max(cur, axis=1, keepdims=True)
        idx = jnp.min(jnp.where(cur == mx, niof, 1e9), axis=1, keepdims=True)
        first = niof == idx
        bmask = bmask + first.astype(jnp.float32) * (mx > -1e20).astype(jnp.float32)
        cur = jnp.where(first, -1e38, cur)

    tok_ref[...] = (_dot(bmask, e_ref[...]) - 1.0) * 1e30

    j0 = jnp.maximum(i - 2, 0)
    for g in range(G):
        qg = qr[:, g * DH:(g + 1) * DH]

        def far(j, carry):
            m_s, l_s, a_s = carry
            s = _dot(qg, kr_ref[pl.ds(j * KB, KB), :], trans_b=True) * SCALE
            scm = s + tok_ref[:, pl.ds(j * KB, KB)]
            m_n = jnp.maximum(m_s, jnp.max(scm, axis=1, keepdims=True))
            alpha = jnp.exp(m_s - m_n)
            p = jnp.exp(scm - m_n)
            l_s = l_s * alpha + jnp.sum(p, axis=1, keepdims=True)
            a_s = a_s * alpha + _dot(p, vb_ref[pl.ds(j * KB, KB), :])
            return m_n, l_s, a_s

        finit = (jnp.full((QB, 1), NEG), jnp.zeros((QB, 1)),
                 jnp.zeros((QB, DH)))
        m_s, l_s, a_s = jax.lax.fori_loop(0, j0, far, finit)

        def near(j, carry):
            m_s, l_s, a_s, m_w, l_w, a_w = carry
            d = i - j
            s = _dot(qg, kr_ref[pl.ds(j * KB, KB), :], trans_b=True) * SCALE
            vb = vb_ref[pl.ds(j * KB, KB), :]
            scm = (s + tok_ref[:, pl.ds(j * KB, KB)]
                   + cmn_ref[pl.ds(d * QB, QB), :])
            m_n = jnp.maximum(m_s, jnp.max(scm, axis=1, keepdims=True))
            alpha = jnp.exp(m_s - m_n)
            p = jnp.exp(scm - m_n)
            l_s = l_s * alpha + jnp.sum(p, axis=1, keepdims=True)
            a_s = a_s * alpha + _dot(p, vb)
            scw = s + wm_ref[pl.ds(d * QB, QB), :]
            mw_n = jnp.maximum(m_w, jnp.max(scw, axis=1, keepdims=True))
            aw = jnp.exp(m_w - mw_n)
            pw = jnp.exp(scw - mw_n)
            l_w = l_w * aw + jnp.sum(pw, axis=1, keepdims=True)
            a_w = a_w * aw + _dot(pw, vb)
            return m_n, l_s, a_s, mw_n, l_w, a_w

        ninit = (m_s, l_s, a_s,
                 jnp.full((QB, 1), NEG), jnp.zeros((QB, 1)),
                 jnp.zeros((QB, DH)))
        m_s, l_s, a_s, m_w, l_w, a_w = jax.lax.fori_loop(j0, i + 1, near, ninit)

        o_slc = a_s / jnp.maximum(l_s, 1e-9)
        o_swa = a_w / jnp.maximum(l_w, 1e-9)
        gc = g_ref[0, :, 3 * g:3 * g + 1]
        gs = g_ref[0, :, 3 * g + 1:3 * g + 2]
        gw = g_ref[0, :, 3 * g + 2:3 * g + 3]
        o_ref[0, :, g * DH:(g + 1) * DH] = (gc * ocmp[g] + gs * o_slc
                                            + gw * o_swa)


@jax.jit
def kernel(x, Wq, Wk, Wv, Wg, wk_pool, wv_pool, pe):
    x2 = x.reshape(T, D)
    wall = jnp.zeros((D, 11 * 128), jnp.float32)
    wall = wall.at[:, :768].set(Wq).at[:, 768:1024].set(Wk)
    wall = wall.at[:, 1024:1280].set(Wv).at[:, 1280:1316].set(Wg)

    proj = pl.pallas_call(
        _proj_kernel,
        grid=(NQ, 11),
        in_specs=[pl.BlockSpec((QB, D), lambda i, j: (i, 0)),
                  pl.BlockSpec((D, 128), lambda i, j: (0, j))],
        out_specs=pl.BlockSpec((QB, 128), lambda i, j: (i, j)),
        out_shape=jax.ShapeDtypeStruct((T, 11 * 128), jnp.float32),
    )(x2, wall)

    q = proj[:, :768]
    k = proj[:, 768:1024]
    v = proj[:, 1024:1280]
    g36 = proj[:, 1280:1316]
    qh = q.reshape(T, HKV, G * DH).transpose(1, 0, 2)
    kh = k.reshape(T, HKV, DH).transpose(1, 0, 2)
    vh = v.reshape(T, HKV, DH).transpose(1, 0, 2)
    garr = jnp.zeros((HKV, T, 16), jnp.float32).at[:, :, :9].set(
        g36.reshape(T, HKV, 9).transpose(1, 0, 2))

    wkvec = jnp.repeat(wk_pool.T, DH, axis=1)
    wvvec = jnp.repeat(wv_pool.T, DH, axis=1)
    pef = pe.transpose(1, 0, 2).reshape(KS, HKV * DH)

    k2 = k.reshape(T // STRIDE, STRIDE, HKV * DH).transpose(1, 0, 2)
    v2 = v.reshape(T // STRIDE, STRIDE, HKV * DH).transpose(1, 0, 2)
    full = lambda shape: pl.BlockSpec(shape, lambda *a: tuple(0 for _ in shape))
    kc, vc = pl.pallas_call(
        _cmp_kernel,
        grid=(1,),
        in_specs=[full((STRIDE, TCP, HKV * DH)), full((STRIDE, TCP, HKV * DH)),
                  full((KS, HKV * DH)), full((KS, HKV * DH)),
                  full((KS, HKV * DH)),
                  full((TCP, TCP)), full((TCP, HKV * DH)), full((TCP, HKV * DH)),
                  full((HKV * DH, HKV * DH))],
        out_specs=[full((TCP, HKV * DH)), full((TCP, HKV * DH))],
        out_shape=[jax.ShapeDtypeStruct((TCP, HKV * DH), jnp.float32),
                   jax.ShapeDtypeStruct((TCP, HKV * DH), jnp.float32)],
    )(k2, v2, wkvec, wvvec, pef, _SH, _CC256, _SC256, _P256)
    kch = kc.reshape(TCP, HKV, DH).transpose(1, 0, 2)
    vch = vc.reshape(TCP, HKV, DH).transpose(1, 0, 2)

    out = pl.pallas_call(
        _main_kernel,
        grid=(HKV, NQ),
        in_specs=[pl.BlockSpec((1, QB, G * DH), lambda h, i: (h, i, 0)),
                  pl.BlockSpec((1, T, DH), lambda h, i: (h, 0, 0)),
                  pl.BlockSpec((1, T, DH), lambda h, i: (h, 0, 0)),
                  pl.BlockSpec((1, TCP, DH), lambda h, i: (h, 0, 0)),
                  pl.BlockSpec((1, TCP, DH), lambda h, i: (h, 0, 0)),
                  pl.BlockSpec((QB, G * DH), lambda h, i: (i, 0)),
                  pl.BlockSpec((QB, G * DH), lambda h, i: (i, 0)),
                  pl.BlockSpec((T, DH), lambda h, i: (0, 0)),
                  pl.BlockSpec((T, DH), lambda h, i: (0, 0)),
                  pl.BlockSpec((G * DH, G * DH), lambda h, i: (0, 0)),
                  pl.BlockSpec((DH, DH), lambda h, i: (0, 0)),
                  pl.BlockSpec((TCP, NBLK), lambda h, i: (0, 0)),
                  pl.BlockSpec((QB, TCP), lambda h, i: (i, 0)),
                  pl.BlockSpec((NBLK, T), lambda h, i: (0, 0)),
                  pl.BlockSpec((3 * QB, KB), lambda h, i: (0, 0)),
                  pl.BlockSpec((3 * QB, KB), lambda h, i: (0, 0)),
                  pl.BlockSpec((1, QB, 16), lambda h, i: (h, i, 0))],
        out_specs=pl.BlockSpec((1, QB, G * DH), lambda h, i: (h, i, 0)),
        out_shape=jax.ShapeDtypeStruct((HKV, T, G * DH), jnp.float32),
        scratch_shapes=[pltpu.VMEM((QB, T), jnp.float32),
                        pltpu.VMEM((T, DH), jnp.bfloat16),
                        pltpu.VMEM((T, DH), jnp.bfloat16)],
    )(qh, kh, vh, kch, vch, _CQ192, _SQ192, _CK64, _SK64, _P192, _P64,
      _M, _CMADD, _E2048, _WM, _CM, garr)

    return out.transpose(1, 0, 2).reshape(B, T, HQ * DH)


# speedup vs baseline: 1.5950x; 1.1214x over previous
import functools
import math

import jax
import jax.numpy as jnp
import numpy as np
from jax.experimental import pallas as pl
from jax.experimental.pallas import tpu as pltpu

B, T, D = 1, 2048, 768
HQ, HKV = 12, 4
G = HQ // HKV
DH = 64
KS, STRIDE = 32, 16
BS = 64
TOPN = 8
WINDOW = 512
THETA = 10000.0

TC = (T - KS) // STRIDE + 1
TCP = 128
NBLK = T // BS
QB = 256
NQ = T // QB
KB = 256
NEG = -1e30
SCALE = 1.0 / math.sqrt(DH)

def _p_swap(n_heads):
    p1 = np.zeros((DH, DH), np.float32)
    p1[np.arange(32), np.arange(32) + 32] = 1.0
    p1[np.arange(32) + 32, np.arange(32)] = 1.0
    out = np.zeros((n_heads * DH, n_heads * DH), np.float32)
    for h in range(n_heads):
        out[h * DH:(h + 1) * DH, h * DH:(h + 1) * DH] = p1
    return out


def _rope_tables(pos, n_heads):
    inv = 1.0 / (THETA ** (np.arange(0, DH, 2, dtype=np.float32) / DH))
    ang = pos.astype(np.float32)[:, None] * inv[None, :]
    c = np.cos(ang)
    s = np.sin(ang)
    c64 = np.concatenate([c, c], axis=1)
    s64 = np.concatenate([-s, s], axis=1)
    return np.tile(c64, (1, n_heads)), np.tile(s64, (1, n_heads))


_P64 = _p_swap(1)
_P192 = _p_swap(G)
_P256 = _p_swap(HKV)
_CQ192, _SQ192 = _rope_tables(np.arange(T), G)
_CK64, _SK64 = _rope_tables(np.arange(T), 1)
_pc = np.arange(TCP) * STRIDE
_CC256, _SC256 = _rope_tables(_pc, HKV)

_SH = np.zeros((TCP, TCP), np.float32)
_SH[np.arange(TCP - 1), np.arange(TCP - 1) + 1] = 1.0


_M = np.zeros((TCP, NBLK), np.float32)
for _c in range(TC):
    _M[_c, (_c * STRIDE) // BS] = 1.0


_E2048 = np.zeros((NBLK, T), np.float32)
for _b in range(NBLK):
    _E2048[_b, _b * BS:(_b + 1) * BS] = 1.0


_CMADD = np.full((T, TCP), -1e30, np.float32)
for _c in range(TC):
    _CMADD[_c * STRIDE + KS - 1:, _c] = 0.0


_tr = np.arange(QB)[:, None]
_cc = np.arange(KB)[None, :]
_wm = np.zeros((3 * QB, KB), np.float32)
_cm = np.zeros((3 * QB, KB), np.float32)
for _d in range(3):
    ok = (_cc <= _d * KB + _tr) & (_d * KB + _tr - _cc <= WINDOW)
    _wm[_d * QB:(_d + 1) * QB] = np.where(ok, 0.0, -1e30)
    if _d == 0:
        _cm[_d * QB:(_d + 1) * QB] = np.where(_cc <= _tr, 0.0, -1e30)
_WM = _wm
_CM = _cm


def _dot(a, b, trans_b=False):
    dn = (((1,), (1 if trans_b else 0,)), ((), ()))
    return jax.lax.dot_general(a.astype(jnp.bfloat16), b.astype(jnp.bfloat16),
                               dn, preferred_element_type=jnp.float32)


def _dotx(a, b, trans_b=False):
    dn = (((1,), (1 if trans_b else 0,)), ((), ()))
    return jax.lax.dot_general(a, b, dn, preferred_element_type=jnp.float32,
                               precision=jax.lax.Precision.HIGHEST)


def _bf(x):
    return x.astype(jnp.bfloat16).astype(jnp.float32)


def _proj_kernel(x_ref, w_ref, o_ref):
    j = pl.program_id(1)
    r = _dot(x_ref[...], w_ref[...])
    o_ref[...] = jnp.where(j == 10, jax.nn.sigmoid(r), r)


def _cmp_kernel(k2_ref, v2_ref, wk_ref, wv_ref, pe_ref,
                sh_ref, cc_ref, sc_ref, p256_ref, kc_ref, vc_ref):
    ak = jnp.zeros((TCP, HKV * DH), jnp.float32)
    bk = jnp.zeros((TCP, HKV * DH), jnp.float32)
    av = jnp.zeros((TCP, HKV * DH), jnp.float32)
    bv = jnp.zeros((TCP, HKV * DH), jnp.float32)
    wkb = _bf(wk_ref[...])
    wvb = _bf(wv_ref[...])
    for j in range(STRIDE):
        k2j = k2_ref[j]
        v2j = v2_ref[j]
        ka = _bf(k2j + pe_ref[j, :])
        kb = _bf(k2j + pe_ref[j + STRIDE, :])
        va = _bf(v2j + pe_ref[j, :])
        vb = _bf(v2j + pe_ref[j + STRIDE, :])
        ak += ka * wkb[j, :]
        bk += kb * wkb[j + STRIDE, :]
        av += va * wvb[j, :]
        bv += vb * wvb[j + STRIDE, :]
    kc = ak + _dotx(sh_ref[...], bk)
    vc = av + _dotx(sh_ref[...], bv)
    kc_ref[...] = kc * cc_ref[...] + _dotx(kc, p256_ref[...]) * sc_ref[...]
    vc_ref[...] = vc


def _main_kernel(q_ref, k_ref, v_ref, kc_ref, vc_ref, cq_ref, sq_ref,
                 ck_ref, sk_ref, p192_ref, p64_ref, m_ref, cma_ref,
                 e_ref, wm_ref, cmn_ref, g_ref, o_ref,
                 tok_ref, kr_ref, vb_ref):
    i = pl.program_id(1)

    @pl.when(i == 0)
    def _prep():
        kb = k_ref[0]
        krf = kb * ck_ref[...] + _dotx(kb, p64_ref[...]) * sk_ref[...]
        kr_ref[...] = krf.astype(jnp.bfloat16)
        vb_ref[...] = v_ref[0].astype(jnp.bfloat16)

    q = q_ref[0]
    qr = ((q * cq_ref[...] + _dotx(q, p192_ref[...]) * sq_ref[...]) * SCALE
          ).astype(jnp.bfloat16)

    cmadd = cma_ref[...]
    kc = kc_ref[0]
    vc = vc_ref[0]
    psum = jnp.zeros((QB, TCP), jnp.float32)
    ocmp = []
    for g in range(G):
        qg = qr[:, g * DH:(g + 1) * DH]
        p = jnp.exp(jnp.minimum(_dot(qg, kc, trans_b=True) + cmadd, 80.0))
        denom = jnp.maximum(jnp.sum(p, axis=1, keepdims=True), 1e-9)
        p = p / denom
        ocmp.append(_dot(p, vc))
        psum += p

    bscore = _dot(psum, m_ref[...])
    trow = (i * QB + jax.lax.broadcasted_iota(jnp.int32, (QB, 1), 0))
    qblk = trow // BS
    nio = jax.lax.broadcasted_iota(jnp.int32, (QB, NBLK), 1)
    forced = (nio == 0) | (nio == qblk) | (nio == qblk - 1)
    valid = nio <= qblk
    cur = jnp.where(valid, bscore + forced.astype(jnp.float32) * 1e4, NEG)
    niof = nio.astype(jnp.float32)
    bmask = jnp.zeros((QB, NBLK), jnp.float32)
    for _ in range(TOPN):
        mx = jnp.max(cur, axis=1, keepdims=True)
        idx = jnp.min(jnp.where(cur == mx, niof, 1e9), axis=1, keepdims=True)
        first = niof == idx
        bmask = bmask + first.astype(jnp.float32) * (mx > -1e20).astype(jnp.float32)
        cur = jnp.where(first, -1e38, cur)

    tok_ref[...] = (_dot(bmask, e_ref[...]) - 1.0) * 1e30

    j0 = jnp.maximum(i - 2, 0)
    for g in range(G):
        qg = qr[:, g * DH:(g + 1) * DH]

        def far(j, carry):
            l_s, a_s = carry
            s = _dot(qg, kr_ref[pl.ds(j * KB, KB), :], trans_b=True)
            p = jnp.exp(jnp.minimum(s + tok_ref[:, pl.ds(j * KB, KB)], 80.0))
            l_s = l_s + jnp.sum(p, axis=1, keepdims=True)
            a_s = a_s + _dot(p, vb_ref[pl.ds(j * KB, KB), :])
            return l_s, a_s

        finit = (jnp.zeros((QB, 1)), jnp.zeros((QB, DH)))
        l_s, a_s = jax.lax.fori_loop(0, j0, far, finit)

        def near(j, carry):
            l_s, a_s, l_w, a_w = carry
            d = i - j
            s = _dot(qg, kr_ref[pl.ds(j * KB, KB), :], trans_b=True)
            vb = vb_ref[pl.ds(j * KB, KB), :]
            p = jnp.exp(jnp.minimum(s + tok_ref[:, pl.ds(j * KB, KB)]
                                    + cmn_ref[pl.ds(d * QB, QB), :], 80.0))
            l_s = l_s + jnp.sum(p, axis=1, keepdims=True)
            a_s = a_s + _dot(p, vb)
            pw = jnp.exp(jnp.minimum(s + wm_ref[pl.ds(d * QB, QB), :], 80.0))
            l_w = l_w + jnp.sum(pw, axis=1, keepdims=True)
            a_w = a_w + _dot(pw, vb)
            return l_s, a_s, l_w, a_w

        ninit = (l_s, a_s, jnp.zeros((QB, 1)), jnp.zeros((QB, DH)))
        l_s, a_s, l_w, a_w = jax.lax.fori_loop(j0, i + 1, near, ninit)

        o_slc = a_s / jnp.maximum(l_s, 1e-9)
        o_swa = a_w / jnp.maximum(l_w, 1e-9)
        gc = g_ref[0, :, 3 * g:3 * g + 1]
        gs = g_ref[0, :, 3 * g + 1:3 * g + 2]
        gw = g_ref[0, :, 3 * g + 2:3 * g + 3]
        o_ref[0, :, g * DH:(g + 1) * DH] = (gc * ocmp[g] + gs * o_slc
                                            + gw * o_swa)


@jax.jit
def kernel(x, Wq, Wk, Wv, Wg, wk_pool, wv_pool, pe):
    x2 = x.reshape(T, D)
    wall = jnp.zeros((D, 11 * 128), jnp.float32)
    wall = wall.at[:, :768].set(Wq).at[:, 768:1024].set(Wk)
    wall = wall.at[:, 1024:1280].set(Wv).at[:, 1280:1316].set(Wg)

    proj = pl.pallas_call(
        _proj_kernel,
        grid=(NQ, 11),
        in_specs=[pl.BlockSpec((QB, D), lambda i, j: (i, 0)),
                  pl.BlockSpec((D, 128), lambda i, j: (0, j))],
        out_specs=pl.BlockSpec((QB, 128), lambda i, j: (i, j)),
        out_shape=jax.ShapeDtypeStruct((T, 11 * 128), jnp.float32),
    )(x2, wall)

    q = proj[:, :768]
    k = proj[:, 768:1024]
    v = proj[:, 1024:1280]
    g36 = proj[:, 1280:1316]
    qh = q.reshape(T, HKV, G * DH).transpose(1, 0, 2)
    kh = k.reshape(T, HKV, DH).transpose(1, 0, 2)
    vh = v.reshape(T, HKV, DH).transpose(1, 0, 2)
    garr = jnp.zeros((HKV, T, 16), jnp.float32).at[:, :, :9].set(
        g36.reshape(T, HKV, 9).transpose(1, 0, 2))

    wkvec = jnp.repeat(wk_pool.T, DH, axis=1)
    wvvec = jnp.repeat(wv_pool.T, DH, axis=1)
    pef = pe.transpose(1, 0, 2).reshape(KS, HKV * DH)

    k2 = k.reshape(T // STRIDE, STRIDE, HKV * DH).transpose(1, 0, 2)
    v2 = v.reshape(T // STRIDE, STRIDE, HKV * DH).transpose(1, 0, 2)
    full = lambda shape: pl.BlockSpec(shape, lambda *a: tuple(0 for _ in shape))
    kc, vc = pl.pallas_call(
        _cmp_kernel,
        grid=(1,),
        in_specs=[full((STRIDE, TCP, HKV * DH)), full((STRIDE, TCP, HKV * DH)),
                  full((KS, HKV * DH)), full((KS, HKV * DH)),
                  full((KS, HKV * DH)),
                  full((TCP, TCP)), full((TCP, HKV * DH)), full((TCP, HKV * DH)),
                  full((HKV * DH, HKV * DH))],
        out_specs=[full((TCP, HKV * DH)), full((TCP, HKV * DH))],
        out_shape=[jax.ShapeDtypeStruct((TCP, HKV * DH), jnp.float32),
                   jax.ShapeDtypeStruct((TCP, HKV * DH), jnp.float32)],
    )(k2, v2, wkvec, wvvec, pef, _SH, _CC256, _SC256, _P256)
    kch = kc.reshape(TCP, HKV, DH).transpose(1, 0, 2)
    vch = vc.reshape(TCP, HKV, DH).transpose(1, 0, 2)

    out = pl.pallas_call(
        _main_kernel,
        grid=(HKV, NQ),
        in_specs=[pl.BlockSpec((1, QB, G * DH), lambda h, i: (h, i, 0)),
                  pl.BlockSpec((1, T, DH), lambda h, i: (h, 0, 0)),
                  pl.BlockSpec((1, T, DH), lambda h, i: (h, 0, 0)),
                  pl.BlockSpec((1, TCP, DH), lambda h, i: (h, 0, 0)),
                  pl.BlockSpec((1, TCP, DH), lambda h, i: (h, 0, 0)),
                  pl.BlockSpec((QB, G * DH), lambda h, i: (i, 0)),
                  pl.BlockSpec((QB, G * DH), lambda h, i: (i, 0)),
                  pl.BlockSpec((T, DH), lambda h, i: (0, 0)),
                  pl.BlockSpec((T, DH), lambda h, i: (0, 0)),
                  pl.BlockSpec((G * DH, G * DH), lambda h, i: (0, 0)),
                  pl.BlockSpec((DH, DH), lambda h, i: (0, 0)),
                  pl.BlockSpec((TCP, NBLK), lambda h, i: (0, 0)),
                  pl.BlockSpec((QB, TCP), lambda h, i: (i, 0)),
                  pl.BlockSpec((NBLK, T), lambda h, i: (0, 0)),
                  pl.BlockSpec((3 * QB, KB), lambda h, i: (0, 0)),
                  pl.BlockSpec((3 * QB, KB), lambda h, i: (0, 0)),
                  pl.BlockSpec((1, QB, 16), lambda h, i: (h, i, 0))],
        out_specs=pl.BlockSpec((1, QB, G * DH), lambda h, i: (h, i, 0)),
        out_shape=jax.ShapeDtypeStruct((HKV, T, G * DH), jnp.float32),
        scratch_shapes=[pltpu.VMEM((QB, T), jnp.float32),
                        pltpu.VMEM((T, DH), jnp.bfloat16),
                        pltpu.VMEM((T, DH), jnp.bfloat16)],
    )(qh, kh, vh, kch, vch, _CQ192, _SQ192, _CK64, _SK64, _P192, _P64,
      _M, _CMADD, _E2048, _WM, _CM, garr)

    return out.transpose(1, 0, 2).reshape(B, T, HQ * DH)


# speedup vs baseline: 1.6188x; 1.0149x over previous
import functools
import math

import jax
import jax.numpy as jnp
import numpy as np
from jax.experimental import pallas as pl
from jax.experimental.pallas import tpu as pltpu

B, T, D = 1, 2048, 768
HQ, HKV = 12, 4
G = HQ // HKV
DH = 64
KS, STRIDE = 32, 16
BS = 64
TOPN = 8
WINDOW = 512
THETA = 10000.0

TC = (T - KS) // STRIDE + 1
TCP = 128
NBLK = T // BS
QB = 256
NQ = T // QB
KB = 256
NEG = -1e30
SCALE = 1.0 / math.sqrt(DH)

def _p_swap(n_heads):
    p1 = np.zeros((DH, DH), np.float32)
    p1[np.arange(32), np.arange(32) + 32] = 1.0
    p1[np.arange(32) + 32, np.arange(32)] = 1.0
    out = np.zeros((n_heads * DH, n_heads * DH), np.float32)
    for h in range(n_heads):
        out[h * DH:(h + 1) * DH, h * DH:(h + 1) * DH] = p1
    return out


def _rope_tables(pos, n_heads):
    inv = 1.0 / (THETA ** (np.arange(0, DH, 2, dtype=np.float32) / DH))
    ang = pos.astype(np.float32)[:, None] * inv[None, :]
    c = np.cos(ang)
    s = np.sin(ang)
    c64 = np.concatenate([c, c], axis=1)
    s64 = np.concatenate([-s, s], axis=1)
    return np.tile(c64, (1, n_heads)), np.tile(s64, (1, n_heads))


_P64 = _p_swap(1)
_P192 = _p_swap(G)
_P256 = _p_swap(HKV)
_CQ192, _SQ192 = _rope_tables(np.arange(T), G)
_CK64, _SK64 = _rope_tables(np.arange(T), 1)
_pc = np.arange(TCP) * STRIDE
_CC256, _SC256 = _rope_tables(_pc, HKV)

_SH = np.zeros((TCP, TCP), np.float32)
_SH[np.arange(TCP - 1), np.arange(TCP - 1) + 1] = 1.0


_M = np.zeros((TCP, NBLK), np.float32)
for _c in range(TC):
    _M[_c, (_c * STRIDE) // BS] = 1.0


_E2048 = np.zeros((NBLK, T), np.float32)
for _b in range(NBLK):
    _E2048[_b, _b * BS:(_b + 1) * BS] = 1.0


_CMADD = np.full((T, TCP), -1e30, np.float32)
for _c in range(TC):
    _CMADD[_c * STRIDE + KS - 1:, _c] = 0.0


_tr = np.arange(QB)[:, None]
_cc = np.arange(KB)[None, :]
_wm = np.zeros((3 * QB, KB), np.float32)
_cm = np.zeros((3 * QB, KB), np.float32)
for _d in range(3):
    ok = (_cc <= _d * KB + _tr) & (_d * KB + _tr - _cc <= WINDOW)
    _wm[_d * QB:(_d + 1) * QB] = np.where(ok, 0.0, -1e30)
    if _d == 0:
        _cm[_d * QB:(_d + 1) * QB] = np.where(_cc <= _tr, 0.0, -1e30)
_WM = _wm
_CM = _cm


def _dot(a, b, trans_b=False):
    dn = (((1,), (1 if trans_b else 0,)), ((), ()))
    return jax.lax.dot_general(a.astype(jnp.bfloat16), b.astype(jnp.bfloat16),
                               dn, preferred_element_type=jnp.float32)


def _dotx(a, b, trans_b=False):
    dn = (((1,), (1 if trans_b else 0,)), ((), ()))
    return jax.lax.dot_general(a, b, dn, preferred_element_type=jnp.float32,
                               precision=jax.lax.Precision.HIGHEST)


def _bf(x):
    return x.astype(jnp.bfloat16).astype(jnp.float32)


def _proj_kernel(x_ref, w_ref, o_ref):
    j = pl.program_id(1)
    r = _dot(x_ref[...], w_ref[...])
    o_ref[...] = jnp.where(j == 10, jax.nn.sigmoid(r), r)


def _cmp_kernel(k2_ref, v2_ref, wk_ref, wv_ref, pe_ref,
                sh_ref, cc_ref, sc_ref, p256_ref, kc_ref, vc_ref):
    ak = jnp.zeros((TCP, HKV * DH), jnp.float32)
    bk = jnp.zeros((TCP, HKV * DH), jnp.float32)
    av = jnp.zeros((TCP, HKV * DH), jnp.float32)
    bv = jnp.zeros((TCP, HKV * DH), jnp.float32)
    wkb = _bf(wk_ref[...])
    wvb = _bf(wv_ref[...])
    for j in range(STRIDE):
        k2j = k2_ref[j]
        v2j = v2_ref[j]
        ka = _bf(k2j + pe_ref[j, :])
        kb = _bf(k2j + pe_ref[j + STRIDE, :])
        va = _bf(v2j + pe_ref[j, :])
        vb = _bf(v2j + pe_ref[j + STRIDE, :])
        ak += ka * wkb[j, :]
        bk += kb * wkb[j + STRIDE, :]
        av += va * wvb[j, :]
        bv += vb * wvb[j + STRIDE, :]
    kc = ak + _dotx(sh_ref[...], bk)
    vc = av + _dotx(sh_ref[...], bv)
    kc_ref[...] = kc * cc_ref[...] + _dotx(kc, p256_ref[...]) * sc_ref[...]
    vc_ref[...] = vc


def _main_kernel(q_ref, k_ref, v_ref, kc_ref, vc_ref, cq_ref, sq_ref,
                 ck_ref, sk_ref, p192_ref, p64_ref, m_ref, cma_ref,
                 e_ref, wm_ref, cmn_ref, g_ref, o_ref,
                 tok_ref, kr_ref, vb_ref):
    i = pl.program_id(1)

    @pl.when(i == 0)
    def _prep():
        kb = k_ref[0]
        krf = kb * ck_ref[...] + _dotx(kb, p64_ref[...]) * sk_ref[...]
        kr_ref[...] = krf.astype(jnp.bfloat16)
        vb_ref[...] = v_ref[0].astype(jnp.bfloat16)

    q = q_ref[0]
    cq = cq_ref[...]
    sq = sq_ref[...]
    p64 = p64_ref[...]
    qgs = []
    for g in range(G):
        sl = slice(g * DH, (g + 1) * DH)
        qg = q[:, sl]
        qgs.append(((qg * cq[:, sl] + _dotx(qg, p64) * sq[:, sl]) * SCALE
                    ).astype(jnp.bfloat16))

    cmadd = cma_ref[...]
    kc = kc_ref[0]
    vc = vc_ref[0]
    psum = jnp.zeros((QB, TCP), jnp.float32)
    ocmp = []
    for g in range(G):
        qg = qgs[g]
        p = jnp.exp(jnp.minimum(_dot(qg, kc, trans_b=True) + cmadd, 80.0))
        denom = jnp.maximum(jnp.sum(p, axis=1, keepdims=True), 1e-9)
        p = p / denom
        ocmp.append(_dot(p, vc))
        psum += p

    bscore = _dot(psum, m_ref[...])
    trow = (i * QB + jax.lax.broadcasted_iota(jnp.int32, (QB, 1), 0))
    qblk = trow // BS
    nio = jax.lax.broadcasted_iota(jnp.int32, (QB, NBLK), 1)
    forced = (nio == 0) | (nio == qblk) | (nio == qblk - 1)
    valid = nio <= qblk
    cur = jnp.where(valid, bscore + forced.astype(jnp.float32) * 1e4, NEG)
    niof = nio.astype(jnp.float32)
    bmask = jnp.zeros((QB, NBLK), jnp.float32)
    for _ in range(TOPN):
        mx = jnp.max(cur, axis=1, keepdims=True)
        idx = jnp.min(jnp.where(cur == mx, niof, 1e9), axis=1, keepdims=True)
        first = niof == idx
        bmask = bmask + first.astype(jnp.float32) * (mx > -1e20).astype(jnp.float32)
        cur = jnp.where(first, -1e38, cur)

    tok_ref[...] = (_dot(bmask, e_ref[...]) - 1.0) * 1e30

    j0 = jnp.maximum(i - 2, 0)
    for g in range(G):
        qg = qgs[g]

        def far(j, carry):
            l_s, a_s = carry
            s = _dot(qg, kr_ref[pl.ds(j * KB, KB), :], trans_b=True)
            p = jnp.exp(jnp.minimum(s + tok_ref[:, pl.ds(j * KB, KB)], 80.0))
            l_s = l_s + jnp.sum(p, axis=1, keepdims=True)
            a_s = a_s + _dot(p, vb_ref[pl.ds(j * KB, KB), :])
            return l_s, a_s

        finit = (jnp.zeros((QB, 1)), jnp.zeros((QB, DH)))
        l_s, a_s = jax.lax.fori_loop(0, j0, far, finit)

        def near(j, carry):
            l_s, a_s, l_w, a_w = carry
            d = i - j
            s = _dot(qg, kr_ref[pl.ds(j * KB, KB), :], trans_b=True)
            vb = vb_ref[pl.ds(j * KB, KB), :]
            p = jnp.exp(jnp.minimum(s + tok_ref[:, pl.ds(j * KB, KB)]
                                    + cmn_ref[pl.ds(d * QB, QB), :], 80.0))
            l_s = l_s + jnp.sum(p, axis=1, keepdims=True)
            a_s = a_s + _dot(p, vb)
            pw = jnp.exp(jnp.minimum(s + wm_ref[pl.ds(d * QB, QB), :], 80.0))
            l_w = l_w + jnp.sum(pw, axis=1, keepdims=True)
            a_w = a_w + _dot(pw, vb)
            return l_s, a_s, l_w, a_w

        ninit = (l_s, a_s, jnp.zeros((QB, 1)), jnp.zeros((QB, DH)))
        l_s, a_s, l_w, a_w = jax.lax.fori_loop(j0, i + 1, near, ninit)

        o_slc = a_s / jnp.maximum(l_s, 1e-9)
        o_swa = a_w / jnp.maximum(l_w, 1e-9)
        gc = g_ref[0, :, 3 * g:3 * g + 1]
        gs = g_ref[0, :, 3 * g + 1:3 * g + 2]
        gw = g_ref[0, :, 3 * g + 2:3 * g + 3]
        o_ref[0, :, g * DH:(g + 1) * DH] = (gc * ocmp[g] + gs * o_slc
                                            + gw * o_swa)


@jax.jit
def kernel(x, Wq, Wk, Wv, Wg, wk_pool, wv_pool, pe):
    x2 = x.reshape(T, D)
    wall = jnp.zeros((D, 11 * 128), jnp.float32)
    wall = wall.at[:, :768].set(Wq).at[:, 768:1024].set(Wk)
    wall = wall.at[:, 1024:1280].set(Wv).at[:, 1280:1316].set(Wg)

    proj = pl.pallas_call(
        _proj_kernel,
        grid=(NQ, 11),
        in_specs=[pl.BlockSpec((QB, D), lambda i, j: (i, 0)),
                  pl.BlockSpec((D, 128), lambda i, j: (0, j))],
        out_specs=pl.BlockSpec((QB, 128), lambda i, j: (i, j)),
        out_shape=jax.ShapeDtypeStruct((T, 11 * 128), jnp.float32),
    )(x2, wall)

    q = proj[:, :768]
    k = proj[:, 768:1024]
    v = proj[:, 1024:1280]
    g36 = proj[:, 1280:1316]
    qh = q.reshape(T, HKV, G * DH).transpose(1, 0, 2)
    kh = k.reshape(T, HKV, DH).transpose(1, 0, 2)
    vh = v.reshape(T, HKV, DH).transpose(1, 0, 2)
    garr = jnp.zeros((HKV, T, 16), jnp.float32).at[:, :, :9].set(
        g36.reshape(T, HKV, 9).transpose(1, 0, 2))

    wkvec = jnp.repeat(wk_pool.T, DH, axis=1)
    wvvec = jnp.repeat(wv_pool.T, DH, axis=1)
    pef = pe.transpose(1, 0, 2).reshape(KS, HKV * DH)

    k2 = k.reshape(T // STRIDE, STRIDE, HKV * DH).transpose(1, 0, 2)
    v2 = v.reshape(T // STRIDE, STRIDE, HKV * DH).transpose(1, 0, 2)
    full = lambda shape: pl.BlockSpec(shape, lambda *a: tuple(0 for _ in shape))
    kc, vc = pl.pallas_call(
        _cmp_kernel,
        grid=(1,),
        in_specs=[full((STRIDE, TCP, HKV * DH)), full((STRIDE, TCP, HKV * DH)),
                  full((KS, HKV * DH)), full((KS, HKV * DH)),
                  full((KS, HKV * DH)),
                  full((TCP, TCP)), full((TCP, HKV * DH)), full((TCP, HKV * DH)),
                  full((HKV * DH, HKV * DH))],
        out_specs=[full((TCP, HKV * DH)), full((TCP, HKV * DH))],
        out_shape=[jax.ShapeDtypeStruct((TCP, HKV * DH), jnp.float32),
                   jax.ShapeDtypeStruct((TCP, HKV * DH), jnp.float32)],
    )(k2, v2, wkvec, wvvec, pef, _SH, _CC256, _SC256, _P256)
    kch = kc.reshape(TCP, HKV, DH).transpose(1, 0, 2)
    vch = vc.reshape(TCP, HKV, DH).transpose(1, 0, 2)

    out = pl.pallas_call(
        _main_kernel,
        grid=(HKV, NQ),
        in_specs=[pl.BlockSpec((1, QB, G * DH), lambda h, i: (h, i, 0)),
                  pl.BlockSpec((1, T, DH), lambda h, i: (h, 0, 0)),
                  pl.BlockSpec((1, T, DH), lambda h, i: (h, 0, 0)),
                  pl.BlockSpec((1, TCP, DH), lambda h, i: (h, 0, 0)),
                  pl.BlockSpec((1, TCP, DH), lambda h, i: (h, 0, 0)),
                  pl.BlockSpec((QB, G * DH), lambda h, i: (i, 0)),
                  pl.BlockSpec((QB, G * DH), lambda h, i: (i, 0)),
                  pl.BlockSpec((T, DH), lambda h, i: (0, 0)),
                  pl.BlockSpec((T, DH), lambda h, i: (0, 0)),
                  pl.BlockSpec((G * DH, G * DH), lambda h, i: (0, 0)),
                  pl.BlockSpec((DH, DH), lambda h, i: (0, 0)),
                  pl.BlockSpec((TCP, NBLK), lambda h, i: (0, 0)),
                  pl.BlockSpec((QB, TCP), lambda h, i: (i, 0)),
                  pl.BlockSpec((NBLK, T), lambda h, i: (0, 0)),
                  pl.BlockSpec((3 * QB, KB), lambda h, i: (0, 0)),
                  pl.BlockSpec((3 * QB, KB), lambda h, i: (0, 0)),
                  pl.BlockSpec((1, QB, 16), lambda h, i: (h, i, 0))],
        out_specs=pl.BlockSpec((1, QB, G * DH), lambda h, i: (h, i, 0)),
        out_shape=jax.ShapeDtypeStruct((HKV, T, G * DH), jnp.float32),
        scratch_shapes=[pltpu.VMEM((QB, T), jnp.float32),
                        pltpu.VMEM((T, DH), jnp.bfloat16),
                        pltpu.VMEM((T, DH), jnp.bfloat16)],
    )(qh, kh, vh, kch, vch, _CQ192, _SQ192, _CK64, _SK64, _P192, _P64,
      _M, _CMADD, _E2048, _WM, _CM, garr)

    return out.transpose(1, 0, 2).reshape(B, T, HQ * DH)


# speedup vs baseline: 2.3084x; 1.4260x over previous
import functools
import math

import jax
import jax.numpy as jnp
import numpy as np
from jax.experimental import pallas as pl
from jax.experimental.pallas import tpu as pltpu

B, T, D = 1, 2048, 768
HQ, HKV = 12, 4
G = HQ // HKV
DH = 64
KS, STRIDE = 32, 16
BS = 64
TOPN = 8
WINDOW = 512
THETA = 10000.0

TC = (T - KS) // STRIDE + 1
TCP = 128
NBLK = T // BS
QB = 512
NQ = T // QB
KB = 512
NEG = -1e30
SCALE = 1.0 / math.sqrt(DH)

def _p_swap(n_heads):
    p1 = np.zeros((DH, DH), np.float32)
    p1[np.arange(32), np.arange(32) + 32] = 1.0
    p1[np.arange(32) + 32, np.arange(32)] = 1.0
    out = np.zeros((n_heads * DH, n_heads * DH), np.float32)
    for h in range(n_heads):
        out[h * DH:(h + 1) * DH, h * DH:(h + 1) * DH] = p1
    return out


def _rope_tables(pos, n_heads):
    inv = 1.0 / (THETA ** (np.arange(0, DH, 2, dtype=np.float32) / DH))
    ang = pos.astype(np.float32)[:, None] * inv[None, :]
    c = np.cos(ang)
    s = np.sin(ang)
    c64 = np.concatenate([c, c], axis=1)
    s64 = np.concatenate([-s, s], axis=1)
    return np.tile(c64, (1, n_heads)), np.tile(s64, (1, n_heads))


_P64 = _p_swap(1)
_P192 = _p_swap(G)
_P256 = _p_swap(HKV)
_CQ192, _SQ192 = _rope_tables(np.arange(T), G)
_CK64, _SK64 = _rope_tables(np.arange(T), 1)
_pc = np.arange(TCP) * STRIDE
_CC256, _SC256 = _rope_tables(_pc, HKV)

_SH = np.zeros((TCP, TCP), np.float32)
_SH[np.arange(TCP - 1), np.arange(TCP - 1) + 1] = 1.0


_M = np.zeros((TCP, NBLK), np.float32)
for _c in range(TC):
    _M[_c, (_c * STRIDE) // BS] = 1.0


_E2048 = np.zeros((NBLK, T), np.float32)
for _b in range(NBLK):
    _E2048[_b, _b * BS:(_b + 1) * BS] = 1.0


_CMADD = np.full((T, TCP), -1e30, np.float32)
for _c in range(TC):
    _CMADD[_c * STRIDE + KS - 1:, _c] = 0.0


_tr = np.arange(QB)[:, None]
_cc = np.arange(KB)[None, :]
_wm = np.zeros((2 * QB, KB), np.float32)
_cm = np.zeros((2 * QB, KB), np.float32)
for _d in range(2):
    ok = (_cc <= _d * KB + _tr) & (_d * KB + _tr - _cc <= WINDOW)
    _wm[_d * QB:(_d + 1) * QB] = np.where(ok, 0.0, -1e30)
    if _d == 0:
        _cm[_d * QB:(_d + 1) * QB] = np.where(_cc <= _tr, 0.0, -1e30)
_WM = _wm
_CM = _cm


def _dot(a, b, trans_b=False):
    dn = (((1,), (1 if trans_b else 0,)), ((), ()))
    return jax.lax.dot_general(a.astype(jnp.bfloat16), b.astype(jnp.bfloat16),
                               dn, preferred_element_type=jnp.float32)


def _dotx(a, b, trans_b=False):
    dn = (((1,), (1 if trans_b else 0,)), ((), ()))
    return jax.lax.dot_general(a, b, dn, preferred_element_type=jnp.float32,
                               precision=jax.lax.Precision.HIGHEST)


def _bf(x):
    return x.astype(jnp.bfloat16).astype(jnp.float32)


def _proj_kernel(x_ref, w_ref, o_ref):
    j = pl.program_id(1)
    r = _dot(x_ref[...], w_ref[...])
    o_ref[...] = jnp.where(j == 10, jax.nn.sigmoid(r), r)


def _cmp_kernel(k2_ref, v2_ref, wk_ref, wv_ref, pe_ref,
                sh_ref, cc_ref, sc_ref, p256_ref, kc_ref, vc_ref):
    ak = jnp.zeros((TCP, HKV * DH), jnp.float32)
    bk = jnp.zeros((TCP, HKV * DH), jnp.float32)
    av = jnp.zeros((TCP, HKV * DH), jnp.float32)
    bv = jnp.zeros((TCP, HKV * DH), jnp.float32)
    wkb = _bf(wk_ref[...])
    wvb = _bf(wv_ref[...])
    for j in range(STRIDE):
        k2j = k2_ref[j]
        v2j = v2_ref[j]
        ka = _bf(k2j + pe_ref[j, :])
        kb = _bf(k2j + pe_ref[j + STRIDE, :])
        va = _bf(v2j + pe_ref[j, :])
        vb = _bf(v2j + pe_ref[j + STRIDE, :])
        ak += ka * wkb[j, :]
        bk += kb * wkb[j + STRIDE, :]
        av += va * wvb[j, :]
        bv += vb * wvb[j + STRIDE, :]
    kc = ak + _dotx(sh_ref[...], bk)
    vc = av + _dotx(sh_ref[...], bv)
    kc_ref[...] = kc * cc_ref[...] + _dotx(kc, p256_ref[...]) * sc_ref[...]
    vc_ref[...] = vc


def _main_kernel(q_ref, k_ref, v_ref, kc_ref, vc_ref, cq_ref, sq_ref,
                 ck_ref, sk_ref, p192_ref, p64_ref, m_ref, cma_ref,
                 e_ref, wm_ref, cmn_ref, g_ref, o_ref,
                 tok_ref, kr_ref, vb_ref):
    i = pl.program_id(1)

    @pl.when(i == 0)
    def _prep():
        kb = k_ref[0]
        krf = kb * ck_ref[...] + _dotx(kb, p64_ref[...]) * sk_ref[...]
        kr_ref[...] = krf.astype(jnp.bfloat16)
        vb_ref[...] = v_ref[0].astype(jnp.bfloat16)

    q = q_ref[0]
    cq = cq_ref[...]
    sq = sq_ref[...]
    p64 = p64_ref[...]
    qgs = []
    for g in range(G):
        sl = slice(g * DH, (g + 1) * DH)
        qg = q[:, sl]
        qgs.append(((qg * cq[:, sl] + _dotx(qg, p64) * sq[:, sl]) * SCALE
                    ).astype(jnp.bfloat16))

    cmadd = cma_ref[...]
    kc = kc_ref[0]
    vc = vc_ref[0]
    psum = jnp.zeros((QB, TCP), jnp.float32)
    ocmp = []
    for g in range(G):
        qg = qgs[g]
        p = jnp.exp(jnp.minimum(_dot(qg, kc, trans_b=True) + cmadd, 80.0))
        denom = jnp.maximum(jnp.sum(p, axis=1, keepdims=True), 1e-9)
        p = p / denom
        ocmp.append(_dot(p, vc))
        psum += p

    bscore = _dot(psum, m_ref[...])
    trow = (i * QB + jax.lax.broadcasted_iota(jnp.int32, (QB, 1), 0))
    qblk = trow // BS
    nio = jax.lax.broadcasted_iota(jnp.int32, (QB, NBLK), 1)
    forced = (nio == 0) | (nio == qblk) | (nio == qblk - 1)
    valid = nio <= qblk
    cur = jnp.where(valid, bscore + forced.astype(jnp.float32) * 1e4, NEG)
    niof = nio.astype(jnp.float32)
    bmask = jnp.zeros((QB, NBLK), jnp.float32)
    for _ in range(TOPN):
        mx = jnp.max(cur, axis=1, keepdims=True)
        idx = jnp.min(jnp.where(cur == mx, niof, 1e9), axis=1, keepdims=True)
        first = niof == idx
        bmask = bmask + first.astype(jnp.float32) * (mx > -1e20).astype(jnp.float32)
        cur = jnp.where(first, -1e38, cur)

    tok_ref[...] = (_dot(bmask, e_ref[...]) - 1.0) * 1e30

    j0 = jnp.maximum(i - 1, 0)
    for g in range(G):
        qg = qgs[g]

        def far(j, carry):
            l_s, a_s = carry
            s = _dot(qg, kr_ref[pl.ds(j * KB, KB), :], trans_b=True)
            p = jnp.exp(jnp.minimum(s + tok_ref[:, pl.ds(j * KB, KB)], 80.0))
            l_s = l_s + jnp.sum(p, axis=1, keepdims=True)
            a_s = a_s + _dot(p, vb_ref[pl.ds(j * KB, KB), :])
            return l_s, a_s

        finit = (jnp.zeros((QB, 1)), jnp.zeros((QB, DH)))
        l_s, a_s = jax.lax.fori_loop(0, j0, far, finit)

        def near(j, carry):
            l_s, a_s, l_w, a_w = carry
            d = i - j
            s = _dot(qg, kr_ref[pl.ds(j * KB, KB), :], trans_b=True)
            vb = vb_ref[pl.ds(j * KB, KB), :]
            p = jnp.exp(jnp.minimum(s + tok_ref[:, pl.ds(j * KB, KB)]
                                    + cmn_ref[pl.ds(d * QB, QB), :], 80.0))
            l_s = l_s + jnp.sum(p, axis=1, keepdims=True)
            a_s = a_s + _dot(p, vb)
            pw = jnp.exp(jnp.minimum(s + wm_ref[pl.ds(d * QB, QB), :], 80.0))
            l_w = l_w + jnp.sum(pw, axis=1, keepdims=True)
            a_w = a_w + _dot(pw, vb)
            return l_s, a_s, l_w, a_w

        ninit = (l_s, a_s, jnp.zeros((QB, 1)), jnp.zeros((QB, DH)))
        l_s, a_s, l_w, a_w = jax.lax.fori_loop(j0, i + 1, near, ninit)

        o_slc = a_s / jnp.maximum(l_s, 1e-9)
        o_swa = a_w / jnp.maximum(l_w, 1e-9)
        gc = g_ref[0, :, 3 * g:3 * g + 1]
        gs = g_ref[0, :, 3 * g + 1:3 * g + 2]
        gw = g_ref[0, :, 3 * g + 2:3 * g + 3]
        o_ref[0, :, g * DH:(g + 1) * DH] = (gc * ocmp[g] + gs * o_slc
                                            + gw * o_swa)


@jax.jit
def kernel(x, Wq, Wk, Wv, Wg, wk_pool, wv_pool, pe):
    x2 = x.reshape(T, D)
    wall = jnp.zeros((D, 11 * 128), jnp.float32)
    wall = wall.at[:, :768].set(Wq).at[:, 768:1024].set(Wk)
    wall = wall.at[:, 1024:1280].set(Wv).at[:, 1280:1316].set(Wg)

    proj = pl.pallas_call(
        _proj_kernel,
        grid=(NQ, 11),
        in_specs=[pl.BlockSpec((QB, D), lambda i, j: (i, 0)),
                  pl.BlockSpec((D, 128), lambda i, j: (0, j))],
        out_specs=pl.BlockSpec((QB, 128), lambda i, j: (i, j)),
        out_shape=jax.ShapeDtypeStruct((T, 11 * 128), jnp.float32),
    )(x2, wall)

    q = proj[:, :768]
    k = proj[:, 768:1024]
    v = proj[:, 1024:1280]
    g36 = proj[:, 1280:1316]
    qh = q.reshape(T, HKV, G * DH).transpose(1, 0, 2)
    kh = k.reshape(T, HKV, DH).transpose(1, 0, 2)
    vh = v.reshape(T, HKV, DH).transpose(1, 0, 2)
    garr = jnp.zeros((HKV, T, 16), jnp.float32).at[:, :, :9].set(
        g36.reshape(T, HKV, 9).transpose(1, 0, 2))

    wkvec = jnp.repeat(wk_pool.T, DH, axis=1)
    wvvec = jnp.repeat(wv_pool.T, DH, axis=1)
    pef = pe.transpose(1, 0, 2).reshape(KS, HKV * DH)

    k2 = k.reshape(T // STRIDE, STRIDE, HKV * DH).transpose(1, 0, 2)
    v2 = v.reshape(T // STRIDE, STRIDE, HKV * DH).transpose(1, 0, 2)
    full = lambda shape: pl.BlockSpec(shape, lambda *a: tuple(0 for _ in shape))
    kc, vc = pl.pallas_call(
        _cmp_kernel,
        grid=(1,),
        in_specs=[full((STRIDE, TCP, HKV * DH)), full((STRIDE, TCP, HKV * DH)),
                  full((KS, HKV * DH)), full((KS, HKV * DH)),
                  full((KS, HKV * DH)),
                  full((TCP, TCP)), full((TCP, HKV * DH)), full((TCP, HKV * DH)),
                  full((HKV * DH, HKV * DH))],
        out_specs=[full((TCP, HKV * DH)), full((TCP, HKV * DH))],
        out_shape=[jax.ShapeDtypeStruct((TCP, HKV * DH), jnp.float32),
                   jax.ShapeDtypeStruct((TCP, HKV * DH), jnp.float32)],
    )(k2, v2, wkvec, wvvec, pef, _SH, _CC256, _SC256, _P256)
    kch = kc.reshape(TCP, HKV, DH).transpose(1, 0, 2)
    vch = vc.reshape(TCP, HKV, DH).transpose(1, 0, 2)

    out = pl.pallas_call(
        _main_kernel,
        grid=(HKV, NQ),
        in_specs=[pl.BlockSpec((1, QB, G * DH), lambda h, i: (h, i, 0)),
                  pl.BlockSpec((1, T, DH), lambda h, i: (h, 0, 0)),
                  pl.BlockSpec((1, T, DH), lambda h, i: (h, 0, 0)),
                  pl.BlockSpec((1, TCP, DH), lambda h, i: (h, 0, 0)),
                  pl.BlockSpec((1, TCP, DH), lambda h, i: (h, 0, 0)),
                  pl.BlockSpec((QB, G * DH), lambda h, i: (i, 0)),
                  pl.BlockSpec((QB, G * DH), lambda h, i: (i, 0)),
                  pl.BlockSpec((T, DH), lambda h, i: (0, 0)),
                  pl.BlockSpec((T, DH), lambda h, i: (0, 0)),
                  pl.BlockSpec((G * DH, G * DH), lambda h, i: (0, 0)),
                  pl.BlockSpec((DH, DH), lambda h, i: (0, 0)),
                  pl.BlockSpec((TCP, NBLK), lambda h, i: (0, 0)),
                  pl.BlockSpec((QB, TCP), lambda h, i: (i, 0)),
                  pl.BlockSpec((NBLK, T), lambda h, i: (0, 0)),
                  pl.BlockSpec((2 * QB, KB), lambda h, i: (0, 0)),
                  pl.BlockSpec((2 * QB, KB), lambda h, i: (0, 0)),
                  pl.BlockSpec((1, QB, 16), lambda h, i: (h, i, 0))],
        out_specs=pl.BlockSpec((1, QB, G * DH), lambda h, i: (h, i, 0)),
        out_shape=jax.ShapeDtypeStruct((HKV, T, G * DH), jnp.float32),
        scratch_shapes=[pltpu.VMEM((QB, T), jnp.float32),
                        pltpu.VMEM((T, DH), jnp.bfloat16),
                        pltpu.VMEM((T, DH), jnp.bfloat16)],
    )(qh, kh, vh, kch, vch, _CQ192, _SQ192, _CK64, _SK64, _P192, _P64,
      _M, _CMADD, _E2048, _WM, _CM, garr)

    return out.transpose(1, 0, 2).reshape(B, T, HQ * DH)


# speedup vs baseline: 2.3774x; 1.0299x over previous
import functools
import math

import jax
import jax.numpy as jnp
import numpy as np
from jax.experimental import pallas as pl
from jax.experimental.pallas import tpu as pltpu

B, T, D = 1, 2048, 768
HQ, HKV = 12, 4
G = HQ // HKV
DH = 64
KS, STRIDE = 32, 16
BS = 64
TOPN = 8
WINDOW = 512
THETA = 10000.0

TC = (T - KS) // STRIDE + 1
TCP = 128
NBLK = T // BS
QB = 1024
NQ = T // QB
KB = 1024
NEG = -1e30
SCALE = 1.0 / math.sqrt(DH)

def _p_swap(n_heads):
    p1 = np.zeros((DH, DH), np.float32)
    p1[np.arange(32), np.arange(32) + 32] = 1.0
    p1[np.arange(32) + 32, np.arange(32)] = 1.0
    out = np.zeros((n_heads * DH, n_heads * DH), np.float32)
    for h in range(n_heads):
        out[h * DH:(h + 1) * DH, h * DH:(h + 1) * DH] = p1
    return out


def _rope_tables(pos, n_heads):
    inv = 1.0 / (THETA ** (np.arange(0, DH, 2, dtype=np.float32) / DH))
    ang = pos.astype(np.float32)[:, None] * inv[None, :]
    c = np.cos(ang)
    s = np.sin(ang)
    c64 = np.concatenate([c, c], axis=1)
    s64 = np.concatenate([-s, s], axis=1)
    return np.tile(c64, (1, n_heads)), np.tile(s64, (1, n_heads))


_P64 = _p_swap(1)
_P192 = _p_swap(G)
_P256 = _p_swap(HKV)
_CQ192, _SQ192 = _rope_tables(np.arange(T), G)
_CK64, _SK64 = _rope_tables(np.arange(T), 1)
_pc = np.arange(TCP) * STRIDE
_CC256, _SC256 = _rope_tables(_pc, HKV)

_SH = np.zeros((TCP, TCP), np.float32)
_SH[np.arange(TCP - 1), np.arange(TCP - 1) + 1] = 1.0


_M = np.zeros((TCP, NBLK), np.float32)
for _c in range(TC):
    _M[_c, (_c * STRIDE) // BS] = 1.0


_E2048 = np.zeros((NBLK, T), np.float32)
for _b in range(NBLK):
    _E2048[_b, _b * BS:(_b + 1) * BS] = 1.0


_CMADD = np.full((T, TCP), -1e30, np.float32)
for _c in range(TC):
    _CMADD[_c * STRIDE + KS - 1:, _c] = 0.0


_tr = np.arange(QB)[:, None]
_cc = np.arange(KB)[None, :]
_wm = np.zeros((2 * QB, KB), np.float32)
_cm = np.zeros((2 * QB, KB), np.float32)
for _d in range(2):
    ok = (_cc <= _d * KB + _tr) & (_d * KB + _tr - _cc <= WINDOW)
    _wm[_d * QB:(_d + 1) * QB] = np.where(ok, 0.0, -1e30)
    if _d == 0:
        _cm[_d * QB:(_d + 1) * QB] = np.where(_cc <= _tr, 0.0, -1e30)
_WM = _wm
_CM = _cm


def _dot(a, b, trans_b=False):
    dn = (((1,), (1 if trans_b else 0,)), ((), ()))
    return jax.lax.dot_general(a.astype(jnp.bfloat16), b.astype(jnp.bfloat16),
                               dn, preferred_element_type=jnp.float32)


def _dotx(a, b, trans_b=False):
    dn = (((1,), (1 if trans_b else 0,)), ((), ()))
    return jax.lax.dot_general(a, b, dn, preferred_element_type=jnp.float32,
                               precision=jax.lax.Precision.HIGHEST)


def _bf(x):
    return x.astype(jnp.bfloat16).astype(jnp.float32)


def _proj_kernel(x_ref, w_ref, o_ref):
    j = pl.program_id(1)
    r = _dot(x_ref[...], w_ref[...])
    o_ref[...] = jnp.where(j == 10, jax.nn.sigmoid(r), r)


def _cmp_kernel(k2_ref, v2_ref, wk_ref, wv_ref, pe_ref,
                sh_ref, cc_ref, sc_ref, p256_ref, kc_ref, vc_ref):
    ak = jnp.zeros((TCP, HKV * DH), jnp.float32)
    bk = jnp.zeros((TCP, HKV * DH), jnp.float32)
    av = jnp.zeros((TCP, HKV * DH), jnp.float32)
    bv = jnp.zeros((TCP, HKV * DH), jnp.float32)
    wkb = _bf(wk_ref[...])
    wvb = _bf(wv_ref[...])
    for j in range(STRIDE):
        k2j = k2_ref[j]
        v2j = v2_ref[j]
        ka = _bf(k2j + pe_ref[j, :])
        kb = _bf(k2j + pe_ref[j + STRIDE, :])
        va = _bf(v2j + pe_ref[j, :])
        vb = _bf(v2j + pe_ref[j + STRIDE, :])
        ak += ka * wkb[j, :]
        bk += kb * wkb[j + STRIDE, :]
        av += va * wvb[j, :]
        bv += vb * wvb[j + STRIDE, :]
    kc = ak + _dotx(sh_ref[...], bk)
    vc = av + _dotx(sh_ref[...], bv)
    kc_ref[...] = kc * cc_ref[...] + _dotx(kc, p256_ref[...]) * sc_ref[...]
    vc_ref[...] = vc


def _main_kernel(q_ref, k_ref, v_ref, kc_ref, vc_ref, cq_ref, sq_ref,
                 ck_ref, sk_ref, p192_ref, p64_ref, m_ref, cma_ref,
                 e_ref, wm_ref, cmn_ref, g_ref, o_ref,
                 tok_ref, kr_ref, vb_ref):
    i = pl.program_id(1)

    @pl.when(i == 0)
    def _prep():
        kb = k_ref[0]
        krf = kb * ck_ref[...] + _dotx(kb, p64_ref[...]) * sk_ref[...]
        kr_ref[...] = krf.astype(jnp.bfloat16)
        vb_ref[...] = v_ref[0].astype(jnp.bfloat16)

    q = q_ref[0]
    cq = cq_ref[...]
    sq = sq_ref[...]
    p64 = p64_ref[...]
    qgs = []
    for g in range(G):
        sl = slice(g * DH, (g + 1) * DH)
        qg = q[:, sl]
        qgs.append(((qg * cq[:, sl] + _dotx(qg, p64) * sq[:, sl]) * SCALE
                    ).astype(jnp.bfloat16))

    cmadd = cma_ref[...]
    kc = kc_ref[0]
    vc = vc_ref[0]
    psum = jnp.zeros((QB, TCP), jnp.float32)
    ocmp = []
    for g in range(G):
        qg = qgs[g]
        p = jnp.exp(jnp.minimum(_dot(qg, kc, trans_b=True) + cmadd, 80.0))
        denom = jnp.maximum(jnp.sum(p, axis=1, keepdims=True), 1e-9)
        p = p / denom
        ocmp.append(_dot(p, vc))
        psum += p

    bscore = _dot(psum, m_ref[...])
    trow = (i * QB + jax.lax.broadcasted_iota(jnp.int32, (QB, 1), 0))
    qblk = trow // BS
    nio = jax.lax.broadcasted_iota(jnp.int32, (QB, NBLK), 1)
    forced = (nio == 0) | (nio == qblk) | (nio == qblk - 1)
    valid = nio <= qblk
    cur = jnp.where(valid, bscore + forced.astype(jnp.float32) * 1e4, NEG)
    niof = nio.astype(jnp.float32)
    bmask = jnp.zeros((QB, NBLK), jnp.float32)
    for _ in range(TOPN):
        mx = jnp.max(cur, axis=1, keepdims=True)
        idx = jnp.min(jnp.where(cur == mx, niof, 1e9), axis=1, keepdims=True)
        first = niof == idx
        bmask = bmask + first.astype(jnp.float32) * (mx > -1e20).astype(jnp.float32)
        cur = jnp.where(first, -1e38, cur)

    tok_ref[...] = (_dot(bmask, e_ref[...]) - 1.0) * 1e30

    j0 = jnp.maximum(i - 1, 0)
    for g in range(G):
        qg = qgs[g]

        def far(j, carry):
            l_s, a_s = carry
            s = _dot(qg, kr_ref[pl.ds(j * KB, KB), :], trans_b=True)
            p = jnp.exp(jnp.minimum(s + tok_ref[:, pl.ds(j * KB, KB)], 80.0))
            l_s = l_s + jnp.sum(p, axis=1, keepdims=True)
            a_s = a_s + _dot(p, vb_ref[pl.ds(j * KB, KB), :])
            return l_s, a_s

        finit = (jnp.zeros((QB, 1)), jnp.zeros((QB, DH)))
        l_s, a_s = jax.lax.fori_loop(0, j0, far, finit)

        def near(j, carry):
            l_s, a_s, l_w, a_w = carry
            d = i - j
            s = _dot(qg, kr_ref[pl.ds(j * KB, KB), :], trans_b=True)
            vb = vb_ref[pl.ds(j * KB, KB), :]
            p = jnp.exp(jnp.minimum(s + tok_ref[:, pl.ds(j * KB, KB)]
                                    + cmn_ref[pl.ds(d * QB, QB), :], 80.0))
            l_s = l_s + jnp.sum(p, axis=1, keepdims=True)
            a_s = a_s + _dot(p, vb)
            pw = jnp.exp(jnp.minimum(s + wm_ref[pl.ds(d * QB, QB), :], 80.0))
            l_w = l_w + jnp.sum(pw, axis=1, keepdims=True)
            a_w = a_w + _dot(pw, vb)
            return l_s, a_s, l_w, a_w

        ninit = (l_s, a_s, jnp.zeros((QB, 1)), jnp.zeros((QB, DH)))
        l_s, a_s, l_w, a_w = jax.lax.fori_loop(j0, i + 1, near, ninit)

        o_slc = a_s / jnp.maximum(l_s, 1e-9)
        o_swa = a_w / jnp.maximum(l_w, 1e-9)
        gc = g_ref[0, :, 3 * g:3 * g + 1]
        gs = g_ref[0, :, 3 * g + 1:3 * g + 2]
        gw = g_ref[0, :, 3 * g + 2:3 * g + 3]
        o_ref[0, :, g * DH:(g + 1) * DH] = (gc * ocmp[g] + gs * o_slc
                                            + gw * o_swa)


@jax.jit
def kernel(x, Wq, Wk, Wv, Wg, wk_pool, wv_pool, pe):
    x2 = x.reshape(T, D)
    wall = jnp.zeros((D, 11 * 128), jnp.float32)
    wall = wall.at[:, :768].set(Wq).at[:, 768:1024].set(Wk)
    wall = wall.at[:, 1024:1280].set(Wv).at[:, 1280:1316].set(Wg)

    proj = pl.pallas_call(
        _proj_kernel,
        grid=(NQ, 11),
        in_specs=[pl.BlockSpec((QB, D), lambda i, j: (i, 0)),
                  pl.BlockSpec((D, 128), lambda i, j: (0, j))],
        out_specs=pl.BlockSpec((QB, 128), lambda i, j: (i, j)),
        out_shape=jax.ShapeDtypeStruct((T, 11 * 128), jnp.float32),
    )(x2, wall)

    q = proj[:, :768]
    k = proj[:, 768:1024]
    v = proj[:, 1024:1280]
    g36 = proj[:, 1280:1316]
    qh = q.reshape(T, HKV, G * DH).transpose(1, 0, 2)
    kh = k.reshape(T, HKV, DH).transpose(1, 0, 2)
    vh = v.reshape(T, HKV, DH).transpose(1, 0, 2)
    garr = jnp.zeros((HKV, T, 16), jnp.float32).at[:, :, :9].set(
        g36.reshape(T, HKV, 9).transpose(1, 0, 2))

    wkvec = jnp.repeat(wk_pool.T, DH, axis=1)
    wvvec = jnp.repeat(wv_pool.T, DH, axis=1)
    pef = pe.transpose(1, 0, 2).reshape(KS, HKV * DH)

    k2 = k.reshape(T // STRIDE, STRIDE, HKV * DH).transpose(1, 0, 2)
    v2 = v.reshape(T // STRIDE, STRIDE, HKV * DH).transpose(1, 0, 2)
    full = lambda shape: pl.BlockSpec(shape, lambda *a: tuple(0 for _ in shape))
    kc, vc = pl.pallas_call(
        _cmp_kernel,
        grid=(1,),
        in_specs=[full((STRIDE, TCP, HKV * DH)), full((STRIDE, TCP, HKV * DH)),
                  full((KS, HKV * DH)), full((KS, HKV * DH)),
                  full((KS, HKV * DH)),
                  full((TCP, TCP)), full((TCP, HKV * DH)), full((TCP, HKV * DH)),
                  full((HKV * DH, HKV * DH))],
        out_specs=[full((TCP, HKV * DH)), full((TCP, HKV * DH))],
        out_shape=[jax.ShapeDtypeStruct((TCP, HKV * DH), jnp.float32),
                   jax.ShapeDtypeStruct((TCP, HKV * DH), jnp.float32)],
    )(k2, v2, wkvec, wvvec, pef, _SH, _CC256, _SC256, _P256)
    kch = kc.reshape(TCP, HKV, DH).transpose(1, 0, 2)
    vch = vc.reshape(TCP, HKV, DH).transpose(1, 0, 2)

    out = pl.pallas_call(
        _main_kernel,
        grid=(HKV, NQ),
        in_specs=[pl.BlockSpec((1, QB, G * DH), lambda h, i: (h, i, 0)),
                  pl.BlockSpec((1, T, DH), lambda h, i: (h, 0, 0)),
                  pl.BlockSpec((1, T, DH), lambda h, i: (h, 0, 0)),
                  pl.BlockSpec((1, TCP, DH), lambda h, i: (h, 0, 0)),
                  pl.BlockSpec((1, TCP, DH), lambda h, i: (h, 0, 0)),
                  pl.BlockSpec((QB, G * DH), lambda h, i: (i, 0)),
                  pl.BlockSpec((QB, G * DH), lambda h, i: (i, 0)),
                  pl.BlockSpec((T, DH), lambda h, i: (0, 0)),
                  pl.BlockSpec((T, DH), lambda h, i: (0, 0)),
                  pl.BlockSpec((G * DH, G * DH), lambda h, i: (0, 0)),
                  pl.BlockSpec((DH, DH), lambda h, i: (0, 0)),
                  pl.BlockSpec((TCP, NBLK), lambda h, i: (0, 0)),
                  pl.BlockSpec((QB, TCP), lambda h, i: (i, 0)),
                  pl.BlockSpec((NBLK, T), lambda h, i: (0, 0)),
                  pl.BlockSpec((2 * QB, KB), lambda h, i: (0, 0)),
                  pl.BlockSpec((2 * QB, KB), lambda h, i: (0, 0)),
                  pl.BlockSpec((1, QB, 16), lambda h, i: (h, i, 0))],
        out_specs=pl.BlockSpec((1, QB, G * DH), lambda h, i: (h, i, 0)),
        out_shape=jax.ShapeDtypeStruct((HKV, T, G * DH), jnp.float32),
        scratch_shapes=[pltpu.VMEM((QB, T), jnp.float32),
                        pltpu.VMEM((T, DH), jnp.bfloat16),
                        pltpu.VMEM((T, DH), jnp.bfloat16)],
    )(qh, kh, vh, kch, vch, _CQ192, _SQ192, _CK64, _SK64, _P192, _P64,
      _M, _CMADD, _E2048, _WM, _CM, garr)

    return out.transpose(1, 0, 2).reshape(B, T, HQ * DH)


# speedup vs baseline: 2.9648x; 1.2471x over previous
import functools
import math

import jax
import jax.numpy as jnp
import numpy as np
from jax.experimental import pallas as pl
from jax.experimental.pallas import tpu as pltpu

B, T, D = 1, 2048, 768
HQ, HKV = 12, 4
G = HQ // HKV
DH = 64
KS, STRIDE = 32, 16
BS = 64
TOPN = 8
WINDOW = 512
THETA = 10000.0

TC = (T - KS) // STRIDE + 1
TCP = 128
NBLK = T // BS
QB = 512
NQ = T // QB
KB = 512
NEG = -1e30
SCALE = 1.0 / math.sqrt(DH)

def _p_swap(n_heads):
    p1 = np.zeros((DH, DH), np.float32)
    p1[np.arange(32), np.arange(32) + 32] = 1.0
    p1[np.arange(32) + 32, np.arange(32)] = 1.0
    out = np.zeros((n_heads * DH, n_heads * DH), np.float32)
    for h in range(n_heads):
        out[h * DH:(h + 1) * DH, h * DH:(h + 1) * DH] = p1
    return out


def _rope_tables(pos, n_heads):
    inv = 1.0 / (THETA ** (np.arange(0, DH, 2, dtype=np.float32) / DH))
    ang = pos.astype(np.float32)[:, None] * inv[None, :]
    c = np.cos(ang)
    s = np.sin(ang)
    c64 = np.concatenate([c, c], axis=1)
    s64 = np.concatenate([-s, s], axis=1)
    return np.tile(c64, (1, n_heads)), np.tile(s64, (1, n_heads))


_P64 = _p_swap(1)
_P192 = _p_swap(G)
_P256 = _p_swap(HKV)
_CQ192, _SQ192 = _rope_tables(np.arange(T), G)
_CK64, _SK64 = _rope_tables(np.arange(T), 1)
_pc = np.arange(TCP) * STRIDE
_CC256, _SC256 = _rope_tables(_pc, HKV)
_CQ64, _SQ64 = _CK64, _SK64
_CK128, _SK128 = _rope_tables(np.arange(T), 2)
_P128 = _p_swap(2)

_SH = np.zeros((TCP, TCP), np.float32)
_SH[np.arange(TCP - 1), np.arange(TCP - 1) + 1] = 1.0


_M = np.zeros((TCP, NBLK), np.float32)
for _c in range(TC):
    _M[_c, (_c * STRIDE) // BS] = 1.0


_E2048 = np.zeros((NBLK, T), np.float32)
for _b in range(NBLK):
    _E2048[_b, _b * BS:(_b + 1) * BS] = 1.0


_CMADD = np.full((T, TCP), -1e30, np.float32)
for _c in range(TC):
    _CMADD[_c * STRIDE + KS - 1:, _c] = 0.0


_tr = np.arange(QB)[:, None]
_cc = np.arange(KB)[None, :]
_wm = np.zeros((2 * QB, KB), np.float32)
_cm = np.zeros((2 * QB, KB), np.float32)
for _d in range(2):
    ok = (_cc <= _d * KB + _tr) & (_d * KB + _tr - _cc <= WINDOW)
    _wm[_d * QB:(_d + 1) * QB] = np.where(ok, 0.0, -1e30)
    if _d == 0:
        _cm[_d * QB:(_d + 1) * QB] = np.where(_cc <= _tr, 0.0, -1e30)
_WM = _wm
_CM = _cm


def _dot(a, b, trans_b=False):
    dn = (((1,), (1 if trans_b else 0,)), ((), ()))
    return jax.lax.dot_general(a.astype(jnp.bfloat16), b.astype(jnp.bfloat16),
                               dn, preferred_element_type=jnp.float32)


def _dotx(a, b, trans_b=False):
    dn = (((1,), (1 if trans_b else 0,)), ((), ()))
    return jax.lax.dot_general(a, b, dn, preferred_element_type=jnp.float32,
                               precision=jax.lax.Precision.HIGHEST)


def _bf(x):
    return x.astype(jnp.bfloat16).astype(jnp.float32)


def _proj_kernel(x_ref, w_ref, o_ref):
    j = pl.program_id(1)
    r = _dot(x_ref[...], w_ref[...])
    o_ref[...] = jnp.where(j == 10, jax.nn.sigmoid(r), r)


def _cmp_kernel(k2_ref, v2_ref, wk_ref, wv_ref, pe_ref,
                sh_ref, cc_ref, sc_ref, p256_ref, kc_ref, vc_ref):
    ak = jnp.zeros((TCP, HKV * DH), jnp.float32)
    bk = jnp.zeros((TCP, HKV * DH), jnp.float32)
    av = jnp.zeros((TCP, HKV * DH), jnp.float32)
    bv = jnp.zeros((TCP, HKV * DH), jnp.float32)
    wkb = _bf(wk_ref[...])
    wvb = _bf(wv_ref[...])
    for j in range(STRIDE):
        k2j = k2_ref[j]
        v2j = v2_ref[j]
        ka = _bf(k2j + pe_ref[j, :])
        kb = _bf(k2j + pe_ref[j + STRIDE, :])
        va = _bf(v2j + pe_ref[j, :])
        vb = _bf(v2j + pe_ref[j + STRIDE, :])
        ak += ka * wkb[j, :]
        bk += kb * wkb[j + STRIDE, :]
        av += va * wvb[j, :]
        bv += vb * wvb[j + STRIDE, :]
    kc = ak + _dotx(sh_ref[...], bk)
    vc = av + _dotx(sh_ref[...], bv)
    kc_ref[...] = kc * cc_ref[...] + _dotx(kc, p256_ref[...]) * sc_ref[...]
    vc_ref[...] = vc


def _main_kernel(q_ref, k_ref, v_ref, kc_ref, vc_ref, cq_ref, sq_ref,
                 ck_ref, sk_ref, p64_ref, m_ref, cma_ref,
                 e_ref, wm_ref, cmn_ref, g_ref, o_ref,
                 tok_ref, kr_ref, vb_ref):
    i = pl.program_id(1)

    @pl.when(i == 0)
    def _prep():
        kb = k_ref[...]
        krf = kb * ck_ref[...] + _dotx(kb, p64_ref[...]) * sk_ref[...]
        kr_ref[...] = krf.astype(jnp.bfloat16)
        vb_ref[...] = v_ref[...].astype(jnp.bfloat16)

    q = q_ref[...]
    cq = cq_ref[...]
    sq = sq_ref[...]
    p1 = p64_ref[:DH, :DH]
    qgs = [[None] * G for _ in range(2)]
    for hh in range(2):
        for g in range(G):
            sl = slice(hh * G * DH + g * DH, hh * G * DH + (g + 1) * DH)
            qg = q[:, sl]
            qgs[hh][g] = ((qg * cq + _dotx(qg, p1) * sq) * SCALE
                          ).astype(jnp.bfloat16)

    trow = (i * QB + jax.lax.broadcasted_iota(jnp.int32, (QB, 1), 0))
    qblk = trow // BS
    nio = jax.lax.broadcasted_iota(jnp.int32, (QB, NBLK), 1)
    forced = ((nio == 0) | (nio == qblk) | (nio == qblk - 1)
              ).astype(jnp.float32) * 1e4
    valid = nio <= qblk
    niof = nio.astype(jnp.float32)
    cmadd = cma_ref[...]
    mmap = m_ref[...]
    ocmp = [[None] * G for _ in range(2)]

    for hh in range(2):
        kc = kc_ref[:, hh * DH:(hh + 1) * DH]
        vc = vc_ref[:, hh * DH:(hh + 1) * DH]
        psum = jnp.zeros((QB, TCP), jnp.float32)
        for g in range(G):
            p = jnp.exp(jnp.minimum(_dot(qgs[hh][g], kc, trans_b=True)
                                    + cmadd, 80.0))
            denom = jnp.maximum(jnp.sum(p, axis=1, keepdims=True), 1e-9)
            p = p / denom
            ocmp[hh][g] = _dot(p, vc)
            psum += p

        cur = jnp.where(valid, _dot(psum, mmap) + forced, NEG)
        bmask = jnp.zeros((QB, NBLK), jnp.float32)
        for _ in range(TOPN):
            mx = jnp.max(cur, axis=1, keepdims=True)
            idx = jnp.min(jnp.where(cur == mx, niof, 1e9),
                          axis=1, keepdims=True)
            first = niof == idx
            bmask = (bmask + first.astype(jnp.float32)
                     * (mx > -1e20).astype(jnp.float32))
            cur = jnp.where(first, -1e38, cur)

        tok_ref[:, hh * T:(hh + 1) * T] = (_dot(bmask, e_ref[...]) - 1.0) * 1e30

    j0 = jnp.maximum(i - 1, 0)
    for hh in range(2):
        ksl = slice(hh * DH, (hh + 1) * DH)
        for g in range(G):
            qg = qgs[hh][g]

            def far(j, carry):
                l_s, a_s = carry
                s = _dot(qg, kr_ref[pl.ds(j * KB, KB), ksl], trans_b=True)
                p = jnp.exp(jnp.minimum(
                    s + tok_ref[:, pl.ds(hh * T + j * KB, KB)], 80.0))
                l_s = l_s + jnp.sum(p, axis=1, keepdims=True)
                a_s = a_s + _dot(p, vb_ref[pl.ds(j * KB, KB), ksl])
                return l_s, a_s

            finit = (jnp.zeros((QB, 1)), jnp.zeros((QB, DH)))
            l_s, a_s = jax.lax.fori_loop(0, j0, far, finit)

            def near(j, carry):
                l_s, a_s, l_w, a_w = carry
                d = i - j
                s = _dot(qg, kr_ref[pl.ds(j * KB, KB), ksl], trans_b=True)
                vb = vb_ref[pl.ds(j * KB, KB), ksl]
                p = jnp.exp(jnp.minimum(
                    s + tok_ref[:, pl.ds(hh * T + j * KB, KB)]
                    + cmn_ref[pl.ds(d * QB, QB), :], 80.0))
                l_s = l_s + jnp.sum(p, axis=1, keepdims=True)
                a_s = a_s + _dot(p, vb)
                pw = jnp.exp(jnp.minimum(s + wm_ref[pl.ds(d * QB, QB), :],
                                         80.0))
                l_w = l_w + jnp.sum(pw, axis=1, keepdims=True)
                a_w = a_w + _dot(pw, vb)
                return l_s, a_s, l_w, a_w

            ninit = (l_s, a_s, jnp.zeros((QB, 1)), jnp.zeros((QB, DH)))
            l_s, a_s, l_w, a_w = jax.lax.fori_loop(j0, i + 1, near, ninit)

            o_slc = a_s / jnp.maximum(l_s, 1e-9)
            o_swa = a_w / jnp.maximum(l_w, 1e-9)
            gi = hh * 9 + 3 * g
            gc = g_ref[0, :, gi:gi + 1]
            gs = g_ref[0, :, gi + 1:gi + 2]
            gw = g_ref[0, :, gi + 2:gi + 3]
            osl = slice(hh * G * DH + g * DH, hh * G * DH + (g + 1) * DH)
            o_ref[:, osl] = gc * ocmp[hh][g] + gs * o_slc + gw * o_swa


@jax.jit
def kernel(x, Wq, Wk, Wv, Wg, wk_pool, wv_pool, pe):
    x2 = x.reshape(T, D)
    wall = jnp.zeros((D, 11 * 128), jnp.float32)
    wall = wall.at[:, :768].set(Wq).at[:, 768:1024].set(Wk)
    wall = wall.at[:, 1024:1280].set(Wv).at[:, 1280:1316].set(Wg)

    proj = pl.pallas_call(
        _proj_kernel,
        grid=(NQ, 11),
        in_specs=[pl.BlockSpec((QB, D), lambda i, j: (i, 0)),
                  pl.BlockSpec((D, 128), lambda i, j: (0, j))],
        out_specs=pl.BlockSpec((QB, 128), lambda i, j: (i, j)),
        out_shape=jax.ShapeDtypeStruct((T, 11 * 128), jnp.float32),
    )(x2, wall)

    k = proj[:, 768:1024]
    v = proj[:, 1024:1280]
    g36 = proj[:, 1280:1316]
    garr = jnp.zeros((2, T, 32), jnp.float32).at[:, :, :18].set(
        g36.reshape(T, 2, 18).transpose(1, 0, 2))

    wkvec = jnp.repeat(wk_pool.T, DH, axis=1)
    wvvec = jnp.repeat(wv_pool.T, DH, axis=1)
    pef = pe.transpose(1, 0, 2).reshape(KS, HKV * DH)

    k2 = k.reshape(T // STRIDE, STRIDE, HKV * DH).transpose(1, 0, 2)
    v2 = v.reshape(T // STRIDE, STRIDE, HKV * DH).transpose(1, 0, 2)
    full = lambda shape: pl.BlockSpec(shape, lambda *a: tuple(0 for _ in shape))
    kc, vc = pl.pallas_call(
        _cmp_kernel,
        grid=(1,),
        in_specs=[full((STRIDE, TCP, HKV * DH)), full((STRIDE, TCP, HKV * DH)),
                  full((KS, HKV * DH)), full((KS, HKV * DH)),
                  full((KS, HKV * DH)),
                  full((TCP, TCP)), full((TCP, HKV * DH)), full((TCP, HKV * DH)),
                  full((HKV * DH, HKV * DH))],
        out_specs=[full((TCP, HKV * DH)), full((TCP, HKV * DH))],
        out_shape=[jax.ShapeDtypeStruct((TCP, HKV * DH), jnp.float32),
                   jax.ShapeDtypeStruct((TCP, HKV * DH), jnp.float32)],
    )(k2, v2, wkvec, wvvec, pef, _SH, _CC256, _SC256, _P256)

    out = pl.pallas_call(
        _main_kernel,
        grid=(2, NQ),
        in_specs=[pl.BlockSpec((QB, 2 * G * DH), lambda p, i: (i, p)),
                  pl.BlockSpec((T, 2 * DH), lambda p, i: (0, 6 + p)),
                  pl.BlockSpec((T, 2 * DH), lambda p, i: (0, 8 + p)),
                  pl.BlockSpec((TCP, 2 * DH), lambda p, i: (0, p)),
                  pl.BlockSpec((TCP, 2 * DH), lambda p, i: (0, p)),
                  pl.BlockSpec((QB, DH), lambda p, i: (i, 0)),
                  pl.BlockSpec((QB, DH), lambda p, i: (i, 0)),
                  pl.BlockSpec((T, 2 * DH), lambda p, i: (0, 0)),
                  pl.BlockSpec((T, 2 * DH), lambda p, i: (0, 0)),
                  pl.BlockSpec((2 * DH, 2 * DH), lambda p, i: (0, 0)),
                  pl.BlockSpec((TCP, NBLK), lambda p, i: (0, 0)),
                  pl.BlockSpec((QB, TCP), lambda p, i: (i, 0)),
                  pl.BlockSpec((NBLK, T), lambda p, i: (0, 0)),
                  pl.BlockSpec((2 * QB, KB), lambda p, i: (0, 0)),
                  pl.BlockSpec((2 * QB, KB), lambda p, i: (0, 0)),
                  pl.BlockSpec((1, QB, 32), lambda p, i: (p, i, 0))],
        out_specs=pl.BlockSpec((QB, 2 * G * DH), lambda p, i: (i, p)),
        out_shape=jax.ShapeDtypeStruct((T, HQ * DH), jnp.float32),
        scratch_shapes=[pltpu.VMEM((QB, 2 * T), jnp.float32),
                        pltpu.VMEM((T, 2 * DH), jnp.bfloat16),
                        pltpu.VMEM((T, 2 * DH), jnp.bfloat16)],
    )(proj, proj, proj, kc, vc, _CQ64, _SQ64, _CK128, _SK128, _P128,
      _M, _CMADD, _E2048, _WM, _CM, garr)

    return out.reshape(B, T, HQ * DH)
